# baseline (speedup 1.0000x reference)
import functools

import jax
import jax.numpy as jnp
from jax import lax
from jax.experimental import pallas as pl
from jax.experimental.pallas import tpu as pltpu
from jax.experimental.pallas import tpu_sc as plsc

N_NODES = 10000
N_EDGES = 320000
H = 128
EA = 16

_INFO = plsc.get_sparse_core_info()
NC = _INFO.num_cores
NS = _INFO.num_subcores
NW = NC * NS
EPW = N_EDGES // NW
CH = 80
NCH = EPW // CH
N_PAD = 10240
ROWS_PER_TILE = N_PAD // NS

_mesh = plsc.VectorSubcoreMesh(core_axis_name="c", subcore_axis_name="s")


HP = H // 2


@functools.partial(
    pl.kernel,
    out_type=(
        jax.ShapeDtypeStruct((N_EDGES, HP), jnp.int32),
        jax.ShapeDtypeStruct((N_EDGES, HP), jnp.int32),
    ),
    mesh=_mesh,
    scratch_types=[
        pltpu.VMEM((NCH, CH), jnp.int32),
        pltpu.VMEM((NCH, CH), jnp.int32),
        pltpu.VMEM((CH, HP), jnp.int32),
        pltpu.VMEM((CH, HP), jnp.int32),
        pltpu.SemaphoreType.DMA,
        pltpu.SemaphoreType.DMA,
    ],
    compiler_params=pltpu.CompilerParams(use_tc_tiling_on_sc=False),
)
def _sc_gather(node_hbm, src3_hbm, dst3_hbm, hs_hbm, hd_hbm,
               idxs_v, idxd_v, rows_s, rows_d, sem_s, sem_d):
    c = lax.axis_index("c")
    s = lax.axis_index("s")
    wid = s * NC + c
    base_ch = wid * NCH
    pltpu.sync_copy(src3_hbm.at[wid], idxs_v)
    pltpu.sync_copy(dst3_hbm.at[wid], idxd_v)

    def body(j, carry):
        ebase = (base_ch + j) * CH
        cp1 = pltpu.async_copy(node_hbm.at[idxs_v.at[j]], rows_s, sem_s)
        cp2 = pltpu.async_copy(node_hbm.at[idxd_v.at[j]], rows_d, sem_d)
        cp1.wait()
        cp2.wait()
        pltpu.sync_copy(rows_s, hs_hbm.at[pl.ds(ebase, CH)])
        pltpu.sync_copy(rows_d, hd_hbm.at[pl.ds(ebase, CH)])
        return carry

    lax.fori_loop(0, NCH, body, 0)


@functools.partial(
    pl.kernel,
    out_type=jax.ShapeDtypeStruct((NC, N_PAD, H), jnp.float32),
    mesh=_mesh,
    scratch_types=[
        pltpu.VMEM((NCH, CH), jnp.int32),
        pltpu.VMEM((CH, H), jnp.float32),
        pltpu.VMEM_SHARED((N_PAD, H), jnp.float32),
    ],
)
def _sc_scatter(ehn_hbm, dst3_hbm, zeros_hbm, out_hbm, idx_v, rows_v, agg_sh):
    c = lax.axis_index("c")
    s = lax.axis_index("s")
    wid = s * NC + c
    base_ch = wid * NCH

    row0 = s * ROWS_PER_TILE
    pltpu.sync_copy(zeros_hbm.at[pl.ds(row0, ROWS_PER_TILE)],
                    agg_sh.at[pl.ds(row0, ROWS_PER_TILE)])
    plsc.subcore_barrier()

    pltpu.sync_copy(dst3_hbm.at[wid], idx_v)

    def body(j, carry):
        ebase = (base_ch + j) * CH
        pltpu.sync_copy(ehn_hbm.at[pl.ds(ebase, CH)], rows_v)
        pltpu.sync_copy(rows_v, agg_sh.at[idx_v.at[j]], add=True)
        return carry

    lax.fori_loop(0, NCH, body, 0)
    plsc.subcore_barrier()


    pltpu.sync_copy(agg_sh.at[pl.ds(row0, ROWS_PER_TILE)],
                    out_hbm.at[c, pl.ds(row0, ROWS_PER_TILE)])


E_BLK = 512


def _unpack_bf16(p):
    even = lax.bitcast_convert_type(p << 16, jnp.float32).astype(jnp.bfloat16)
    odd = lax.bitcast_convert_type(p & jnp.int32(-65536),
                                   jnp.float32).astype(jnp.bfloat16)
    return even, odd


def _edge_body(hs_ref, hd_ref, ea_ref, eh_ref, ae_ref, ao_ref, be_ref,
               bo_ref, ct_ref, w2t_ref, b1_ref, b2_ref, g_ref, bb_ref,
               out_ref):
    hse, hso = _unpack_bf16(hs_ref[...])
    hde, hdo = _unpack_bf16(hd_ref[...])
    x = (jnp.dot(hse, ae_ref[...], preferred_element_type=jnp.float32)
         + jnp.dot(hso, ao_ref[...], preferred_element_type=jnp.float32)
         + jnp.dot(hde, be_ref[...], preferred_element_type=jnp.float32)
         + jnp.dot(hdo, bo_ref[...], preferred_element_type=jnp.float32)
         + jnp.dot(ea_ref[...], ct_ref[...], preferred_element_type=jnp.float32)
         + b1_ref[...])
    h = jnp.maximum(x, 0.0)
    h_bf = h.astype(jnp.bfloat16)
    msg = jnp.dot(h_bf, w2t_ref[...],
                  preferred_element_type=jnp.float32) + b2_ref[...]
    y = eh_ref[...] + msg
    mu = jnp.mean(y, axis=-1, keepdims=True)
    var = jnp.mean((y - mu) ** 2, axis=-1, keepdims=True)
    out_ref[...] = (y - mu) / jnp.sqrt(var + 1e-5) * g_ref[...] + bb_ref[...]


def _edge_mlp(hs, hd, ea, eh, ae, ao, be, bo, ct, w2t, b1, b2, g, bb):
    grid = (N_EDGES // E_BLK,)
    full = lambda shape: pl.BlockSpec(shape, lambda i: (0, 0))
    return pl.pallas_call(
        _edge_body,
        grid=grid,
        in_specs=[
            pl.BlockSpec((E_BLK, HP), lambda i: (i, 0)),
            pl.BlockSpec((E_BLK, HP), lambda i: (i, 0)),
            pl.BlockSpec((E_BLK, EA), lambda i: (i, 0)),
            pl.BlockSpec((E_BLK, H), lambda i: (i, 0)),
            full((HP, 2 * H)),
            full((HP, 2 * H)),
            full((HP, 2 * H)),
            full((HP, 2 * H)),
            full((EA, 2 * H)),
            full((2 * H, H)),
            full((1, 2 * H)),
            full((1, H)),
            full((1, H)),
            full((1, H)),
        ],
        out_specs=pl.BlockSpec((E_BLK, H), lambda i: (i, 0)),
        out_shape=jax.ShapeDtypeStruct((N_EDGES, H), jnp.float32),
        compiler_params=pltpu.CompilerParams(
            dimension_semantics=("arbitrary",)),
    )(hs, hd, ea, eh, ae, ao, be, bo, ct, w2t, b1, b2, g, bb)


N_BLK = 1000


def _node_body(nh_ref, a0_ref, a1_ref, dt_ref, et_ref, w2t_ref, b1_ref,
               b2_ref, g_ref, bb_ref, out_ref):
    agg = (a0_ref[...] + a1_ref[...]).astype(jnp.bfloat16)
    nh_bf = nh_ref[...].astype(jnp.bfloat16)
    x = (jnp.dot(nh_bf, dt_ref[...], preferred_element_type=jnp.float32)
         + jnp.dot(agg, et_ref[...], preferred_element_type=jnp.float32)
         + b1_ref[...])
    h = jnp.maximum(x, 0.0)
    upd = jnp.dot(h, w2t_ref[...], preferred_element_type=jnp.float32) + b2_ref[...]
    y = nh_ref[...] + upd
    mu = jnp.mean(y, axis=-1, keepdims=True)
    var = jnp.mean((y - mu) ** 2, axis=-1, keepdims=True)
    out_ref[...] = (y - mu) / jnp.sqrt(var + 1e-5) * g_ref[...] + bb_ref[...]


def _node_mlp(nh, a0, a1, dt, et, w2t, b1, b2, g, bb):
    grid = (N_NODES // N_BLK,)
    full = lambda shape: pl.BlockSpec(shape, lambda i: (0, 0))
    return pl.pallas_call(
        _node_body,
        grid=grid,
        in_specs=[
            pl.BlockSpec((N_BLK, H), lambda i: (i, 0)),
            pl.BlockSpec((N_BLK, H), lambda i: (i, 0)),
            pl.BlockSpec((N_BLK, H), lambda i: (i, 0)),
            full((H, 2 * H)),
            full((H, 2 * H)),
            full((2 * H, H)),
            full((1, 2 * H)),
            full((1, H)),
            full((1, H)),
            full((1, H)),
        ],
        out_specs=pl.BlockSpec((N_BLK, H), lambda i: (i, 0)),
        out_shape=jax.ShapeDtypeStruct((N_NODES, H), jnp.float32),
        compiler_params=pltpu.CompilerParams(
            dimension_semantics=("arbitrary",)),
    )(nh, a0, a1, dt, et, w2t, b1, b2, g, bb)


def kernel(node_h, edge_h, edge_index, edge_attr,
           W_e1, b_e1, W_e2, b_e2, W_n1, b_n1, W_n2, b_n2,
           ln_e_g, ln_e_b, ln_n_g, ln_n_b):
    ei = edge_index.astype(jnp.int32)
    src3 = ei[0].reshape(NW, NCH, CH)
    dst3 = ei[1].reshape(NW, NCH, CH)

    node_pack = lax.bitcast_convert_type(
        node_h.astype(jnp.bfloat16).reshape(N_NODES, HP, 2), jnp.int32)
    hs, hd = _sc_gather(node_pack, src3, dst3)

    at = W_e1[:, :H].T
    bt = W_e1[:, H:2 * H].T
    ae = at[0::2].astype(jnp.bfloat16)
    ao = at[1::2].astype(jnp.bfloat16)
    be = bt[0::2].astype(jnp.bfloat16)
    bo = bt[1::2].astype(jnp.bfloat16)
    ct = W_e1[:, 2 * H:].T.astype(jnp.bfloat16)
    w2t = W_e2.T.astype(jnp.bfloat16)
    ea_bf = edge_attr.astype(jnp.bfloat16)
    edge_h_new = _edge_mlp(hs, hd, ea_bf, edge_h, ae, ao, be, bo, ct, w2t,
                           b_e1.reshape(1, -1), b_e2.reshape(1, -1),
                           ln_e_g.reshape(1, -1), ln_e_b.reshape(1, -1))

    zeros_pad = jnp.zeros((N_PAD, H), jnp.float32)
    parts = _sc_scatter(edge_h_new, dst3, zeros_pad)
    p0 = parts[0, :N_NODES]
    p1 = parts[1, :N_NODES]

    dt = W_n1[:, :H].T.astype(jnp.bfloat16)
    et = W_n1[:, H:].T.astype(jnp.bfloat16)
    wn2t = W_n2.T.astype(jnp.bfloat16)
    node_h_new = _node_mlp(node_h, p0, p1, dt, et, wn2t,
                           b_n1.reshape(1, -1), b_n2.reshape(1, -1),
                           ln_n_g.reshape(1, -1), ln_n_b.reshape(1, -1))
    return (node_h_new, edge_h_new)


# speedup vs baseline: 1.2185x; 1.2185x over previous
import functools

import jax
import jax.numpy as jnp
from jax import lax
from jax.experimental import pallas as pl
from jax.experimental.pallas import tpu as pltpu
from jax.experimental.pallas import tpu_sc as plsc

N_NODES = 10000
N_EDGES = 320000
H = 128
EA = 16

_INFO = plsc.get_sparse_core_info()
NC = _INFO.num_cores
NS = _INFO.num_subcores
NW = NC * NS
EPW = N_EDGES // NW
CH = 80
NCH = EPW // CH
N_PAD = 10240
ROWS_PER_TILE = N_PAD // NS

_mesh = plsc.VectorSubcoreMesh(core_axis_name="c", subcore_axis_name="s")


HP = H // 2


@functools.partial(
    pl.kernel,
    out_type=jax.ShapeDtypeStruct((N_EDGES, H), jnp.int32),
    mesh=_mesh,
    scratch_types=[
        pltpu.VMEM((NCH, CH), jnp.int32),
        pltpu.VMEM((NCH, CH), jnp.int32),
        pltpu.VMEM((CH, HP), jnp.int32),
        pltpu.VMEM((CH, HP), jnp.int32),
        pltpu.SemaphoreType.DMA,
        pltpu.SemaphoreType.DMA,
    ],
    compiler_params=pltpu.CompilerParams(use_tc_tiling_on_sc=False),
)
def _sc_gather(node_hbm, src3_hbm, dst3_hbm, hsd_hbm,
               idxs_v, idxd_v, rows_s, rows_d, sem_s, sem_d):
    c = lax.axis_index("c")
    s = lax.axis_index("s")
    wid = s * NC + c
    base_ch = wid * NCH
    pltpu.sync_copy(src3_hbm.at[wid], idxs_v)
    pltpu.sync_copy(dst3_hbm.at[wid], idxd_v)

    def body(j, carry):
        ebase = (base_ch + j) * CH
        cp1 = pltpu.async_copy(node_hbm.at[idxs_v.at[j]], rows_s, sem_s)
        cp2 = pltpu.async_copy(node_hbm.at[idxd_v.at[j]], rows_d, sem_d)
        cp1.wait()
        cp2.wait()
        pltpu.sync_copy(rows_s, hsd_hbm.at[pl.ds(ebase, CH), pl.ds(0, HP)])
        pltpu.sync_copy(rows_d, hsd_hbm.at[pl.ds(ebase, CH), pl.ds(HP, HP)])
        return carry

    lax.fori_loop(0, NCH, body, 0)


@functools.partial(
    pl.kernel,
    out_type=jax.ShapeDtypeStruct((NC, N_PAD, H), jnp.float32),
    mesh=_mesh,
    scratch_types=[
        pltpu.VMEM((NCH, CH), jnp.int32),
        pltpu.VMEM((CH, H), jnp.float32),
        pltpu.VMEM_SHARED((N_PAD, H), jnp.float32),
    ],
)
def _sc_scatter(ehn_hbm, dst3_hbm, zeros_hbm, out_hbm, idx_v, rows_v, agg_sh):
    c = lax.axis_index("c")
    s = lax.axis_index("s")
    wid = s * NC + c
    base_ch = wid * NCH

    row0 = s * ROWS_PER_TILE
    pltpu.sync_copy(zeros_hbm.at[pl.ds(row0, ROWS_PER_TILE)],
                    agg_sh.at[pl.ds(row0, ROWS_PER_TILE)])
    plsc.subcore_barrier()

    pltpu.sync_copy(dst3_hbm.at[wid], idx_v)

    def body(j, carry):
        ebase = (base_ch + j) * CH
        pltpu.sync_copy(ehn_hbm.at[pl.ds(ebase, CH)], rows_v)
        pltpu.sync_copy(rows_v, agg_sh.at[idx_v.at[j]], add=True)
        return carry

    lax.fori_loop(0, NCH, body, 0)
    plsc.subcore_barrier()


    pltpu.sync_copy(agg_sh.at[pl.ds(row0, ROWS_PER_TILE)],
                    out_hbm.at[c, pl.ds(row0, ROWS_PER_TILE)])


E_BLK = 512


def _unpack_bf16(p):
    even = lax.bitcast_convert_type(p << 16, jnp.float32).astype(jnp.bfloat16)
    odd = lax.bitcast_convert_type(p & jnp.int32(-65536),
                                   jnp.float32).astype(jnp.bfloat16)
    return even, odd


def _edge_body(hsd_ref, ea_ref, eh_ref, ae_ref, ao_ref, be_ref,
               bo_ref, ct_ref, w2t_ref, b1_ref, b2_ref, g_ref, bb_ref,
               out_ref):
    hsd = hsd_ref[...]
    hse, hso = _unpack_bf16(hsd[:, :HP])
    hde, hdo = _unpack_bf16(hsd[:, HP:])
    x = (jnp.dot(hse, ae_ref[...], preferred_element_type=jnp.float32)
         + jnp.dot(hso, ao_ref[...], preferred_element_type=jnp.float32)
         + jnp.dot(hde, be_ref[...], preferred_element_type=jnp.float32)
         + jnp.dot(hdo, bo_ref[...], preferred_element_type=jnp.float32)
         + jnp.dot(ea_ref[...], ct_ref[...], preferred_element_type=jnp.float32)
         + b1_ref[...])
    h = jnp.maximum(x, 0.0)
    h_bf = h.astype(jnp.bfloat16)
    msg = jnp.dot(h_bf, w2t_ref[...],
                  preferred_element_type=jnp.float32) + b2_ref[...]
    y = eh_ref[...] + msg
    mu = jnp.mean(y, axis=-1, keepdims=True)
    var = jnp.mean((y - mu) ** 2, axis=-1, keepdims=True)
    out_ref[...] = (y - mu) / jnp.sqrt(var + 1e-5) * g_ref[...] + bb_ref[...]


def _edge_mlp(hsd, ea, eh, ae, ao, be, bo, ct, w2t, b1, b2, g, bb):
    grid = (N_EDGES // E_BLK,)
    full = lambda shape: pl.BlockSpec(shape, lambda i: (0, 0))
    return pl.pallas_call(
        _edge_body,
        grid=grid,
        in_specs=[
            pl.BlockSpec((E_BLK, H), lambda i: (i, 0)),
            pl.BlockSpec((E_BLK, EA), lambda i: (i, 0)),
            pl.BlockSpec((E_BLK, H), lambda i: (i, 0)),
            full((HP, 2 * H)),
            full((HP, 2 * H)),
            full((HP, 2 * H)),
            full((HP, 2 * H)),
            full((EA, 2 * H)),
            full((2 * H, H)),
            full((1, 2 * H)),
            full((1, H)),
            full((1, H)),
            full((1, H)),
        ],
        out_specs=pl.BlockSpec((E_BLK, H), lambda i: (i, 0)),
        out_shape=jax.ShapeDtypeStruct((N_EDGES, H), jnp.float32),
        compiler_params=pltpu.CompilerParams(
            dimension_semantics=("arbitrary",)),
    )(hsd, ea, eh, ae, ao, be, bo, ct, w2t, b1, b2, g, bb)


N_BLK = 1000


def _node_body(nh_ref, a0_ref, a1_ref, dt_ref, et_ref, w2t_ref, b1_ref,
               b2_ref, g_ref, bb_ref, out_ref):
    agg = (a0_ref[...] + a1_ref[...]).astype(jnp.bfloat16)
    nh_bf = nh_ref[...].astype(jnp.bfloat16)
    x = (jnp.dot(nh_bf, dt_ref[...], preferred_element_type=jnp.float32)
         + jnp.dot(agg, et_ref[...], preferred_element_type=jnp.float32)
         + b1_ref[...])
    h = jnp.maximum(x, 0.0)
    upd = jnp.dot(h, w2t_ref[...], preferred_element_type=jnp.float32) + b2_ref[...]
    y = nh_ref[...] + upd
    mu = jnp.mean(y, axis=-1, keepdims=True)
    var = jnp.mean((y - mu) ** 2, axis=-1, keepdims=True)
    out_ref[...] = (y - mu) / jnp.sqrt(var + 1e-5) * g_ref[...] + bb_ref[...]


def _node_mlp(nh, a0, a1, dt, et, w2t, b1, b2, g, bb):
    grid = (N_NODES // N_BLK,)
    full = lambda shape: pl.BlockSpec(shape, lambda i: (0, 0))
    return pl.pallas_call(
        _node_body,
        grid=grid,
        in_specs=[
            pl.BlockSpec((N_BLK, H), lambda i: (i, 0)),
            pl.BlockSpec((N_BLK, H), lambda i: (i, 0)),
            pl.BlockSpec((N_BLK, H), lambda i: (i, 0)),
            full((H, 2 * H)),
            full((H, 2 * H)),
            full((2 * H, H)),
            full((1, 2 * H)),
            full((1, H)),
            full((1, H)),
            full((1, H)),
        ],
        out_specs=pl.BlockSpec((N_BLK, H), lambda i: (i, 0)),
        out_shape=jax.ShapeDtypeStruct((N_NODES, H), jnp.float32),
        compiler_params=pltpu.CompilerParams(
            dimension_semantics=("arbitrary",)),
    )(nh, a0, a1, dt, et, w2t, b1, b2, g, bb)


def kernel(node_h, edge_h, edge_index, edge_attr,
           W_e1, b_e1, W_e2, b_e2, W_n1, b_n1, W_n2, b_n2,
           ln_e_g, ln_e_b, ln_n_g, ln_n_b):
    ei = edge_index.astype(jnp.int32)
    src3 = ei[0].reshape(NW, NCH, CH)
    dst3 = ei[1].reshape(NW, NCH, CH)

    node_pack = lax.bitcast_convert_type(
        node_h.astype(jnp.bfloat16).reshape(N_NODES, HP, 2), jnp.int32)
    hsd = _sc_gather(node_pack, src3, dst3)

    at = W_e1[:, :H].T
    bt = W_e1[:, H:2 * H].T
    ae = at[0::2].astype(jnp.bfloat16)
    ao = at[1::2].astype(jnp.bfloat16)
    be = bt[0::2].astype(jnp.bfloat16)
    bo = bt[1::2].astype(jnp.bfloat16)
    ct = W_e1[:, 2 * H:].T.astype(jnp.bfloat16)
    w2t = W_e2.T.astype(jnp.bfloat16)
    ea_bf = edge_attr.astype(jnp.bfloat16)
    edge_h_new = _edge_mlp(hsd, ea_bf, edge_h, ae, ao, be, bo, ct, w2t,
                           b_e1.reshape(1, -1), b_e2.reshape(1, -1),
                           ln_e_g.reshape(1, -1), ln_e_b.reshape(1, -1))

    zeros_pad = jnp.zeros((N_PAD, H), jnp.float32)
    parts = _sc_scatter(edge_h_new, dst3, zeros_pad)
    p0 = parts[0, :N_NODES]
    p1 = parts[1, :N_NODES]

    dt = W_n1[:, :H].T.astype(jnp.bfloat16)
    et = W_n1[:, H:].T.astype(jnp.bfloat16)
    wn2t = W_n2.T.astype(jnp.bfloat16)
    node_h_new = _node_mlp(node_h, p0, p1, dt, et, wn2t,
                           b_n1.reshape(1, -1), b_n2.reshape(1, -1),
                           ln_n_g.reshape(1, -1), ln_n_b.reshape(1, -1))
    return (node_h_new, edge_h_new)


# speedup vs baseline: 1.7118x; 1.4049x over previous
import functools

import jax
import jax.numpy as jnp
from jax import lax
from jax.experimental import pallas as pl
from jax.experimental.pallas import tpu as pltpu
from jax.experimental.pallas import tpu_sc as plsc

N_NODES = 10000
N_EDGES = 320000
H = 128
EA = 16

_INFO = plsc.get_sparse_core_info()
NC = _INFO.num_cores
NS = _INFO.num_subcores
NW = NC * NS
EPW = N_EDGES // NW
CH = 80
NCH = EPW // CH
N_PAD = 10240
ROWS_PER_TILE = N_PAD // NS

_mesh = plsc.VectorSubcoreMesh(core_axis_name="c", subcore_axis_name="s")


HP = H // 2


@functools.partial(
    pl.kernel,
    out_type=jax.ShapeDtypeStruct((N_EDGES, H), jnp.int32),
    mesh=_mesh,
    scratch_types=[
        pltpu.VMEM((NCH, CH), jnp.int32),
        pltpu.VMEM((NCH, CH), jnp.int32),
        pltpu.VMEM((CH, HP), jnp.int32),
        pltpu.VMEM((CH, HP), jnp.int32),
        pltpu.SemaphoreType.DMA,
        pltpu.SemaphoreType.DMA,
    ],
    compiler_params=pltpu.CompilerParams(use_tc_tiling_on_sc=False),
)
def _sc_gather(node_hbm, src3_hbm, dst3_hbm, hsd_hbm,
               idxs_v, idxd_v, rows_s, rows_d, sem_s, sem_d):
    c = lax.axis_index("c")
    s = lax.axis_index("s")
    wid = s * NC + c
    base_ch = wid * NCH
    pltpu.sync_copy(src3_hbm.at[wid], idxs_v)
    pltpu.sync_copy(dst3_hbm.at[wid], idxd_v)

    def body(j, carry):
        ebase = (base_ch + j) * CH
        cp1 = pltpu.async_copy(node_hbm.at[idxs_v.at[j]], rows_s, sem_s)
        cp2 = pltpu.async_copy(node_hbm.at[idxd_v.at[j]], rows_d, sem_d)
        cp1.wait()
        cp2.wait()
        pltpu.sync_copy(rows_s, hsd_hbm.at[pl.ds(ebase, CH), pl.ds(0, HP)])
        pltpu.sync_copy(rows_d, hsd_hbm.at[pl.ds(ebase, CH), pl.ds(HP, HP)])
        return carry

    lax.fori_loop(0, NCH, body, 0)


@functools.partial(
    pl.kernel,
    out_type=jax.ShapeDtypeStruct((NC, N_PAD, H), jnp.float32),
    mesh=_mesh,
    scratch_types=[
        pltpu.VMEM((NCH, CH), jnp.int32),
        pltpu.VMEM((CH, H), jnp.float32),
        pltpu.VMEM_SHARED((N_PAD, H), jnp.float32),
    ],
)
def _sc_scatter(ehn_hbm, dst3_hbm, zeros_hbm, out_hbm, idx_v, rows_v, agg_sh):
    c = lax.axis_index("c")
    s = lax.axis_index("s")
    wid = s * NC + c
    base_ch = wid * NCH

    row0 = s * ROWS_PER_TILE
    pltpu.sync_copy(zeros_hbm.at[pl.ds(row0, ROWS_PER_TILE)],
                    agg_sh.at[pl.ds(row0, ROWS_PER_TILE)])
    plsc.subcore_barrier()

    pltpu.sync_copy(dst3_hbm.at[wid], idx_v)

    def body(j, carry):
        ebase = (base_ch + j) * CH
        pltpu.sync_copy(ehn_hbm.at[pl.ds(ebase, CH)], rows_v)
        pltpu.sync_copy(rows_v, agg_sh.at[idx_v.at[j]], add=True)
        return carry

    lax.fori_loop(0, NCH, body, 0)
    plsc.subcore_barrier()


    pltpu.sync_copy(agg_sh.at[pl.ds(row0, ROWS_PER_TILE)],
                    out_hbm.at[c, pl.ds(row0, ROWS_PER_TILE)])


E_BLK = 1280


def _unpack_bf16(p):
    even = lax.bitcast_convert_type(p << 16, jnp.float32).astype(jnp.bfloat16)
    odd = lax.bitcast_convert_type(p & jnp.int32(-65536),
                                   jnp.float32).astype(jnp.bfloat16)
    return even, odd


def _edge_body(hsd_ref, ea_ref, eh_ref, w1s_ref, ct_ref, w2t_ref,
               b1_ref, b2_ref, g_ref, bb_ref, out_ref):
    hsd = hsd_ref[...]
    hse, hso = _unpack_bf16(hsd[:, :HP])
    hde, hdo = _unpack_bf16(hsd[:, HP:])
    hcat = jnp.concatenate([hse, hso, hde, hdo], axis=1)
    x = (jnp.dot(hcat, w1s_ref[...], preferred_element_type=jnp.float32)
         + jnp.dot(ea_ref[...], ct_ref[...], preferred_element_type=jnp.float32)
         + b1_ref[...])
    h = jnp.maximum(x, 0.0)
    h_bf = h.astype(jnp.bfloat16)
    msg = jnp.dot(h_bf, w2t_ref[...],
                  preferred_element_type=jnp.float32) + b2_ref[...]
    y = eh_ref[...] + msg
    mu = jnp.mean(y, axis=-1, keepdims=True)
    var = jnp.mean((y - mu) ** 2, axis=-1, keepdims=True)
    out_ref[...] = (y - mu) / jnp.sqrt(var + 1e-5) * g_ref[...] + bb_ref[...]


def _edge_mlp(hsd, ea, eh, w1s, ct, w2t, b1, b2, g, bb):
    grid = (N_EDGES // E_BLK,)
    full = lambda shape: pl.BlockSpec(shape, lambda i: (0, 0))
    return pl.pallas_call(
        _edge_body,
        grid=grid,
        in_specs=[
            pl.BlockSpec((E_BLK, H), lambda i: (i, 0)),
            pl.BlockSpec((E_BLK, EA), lambda i: (i, 0)),
            pl.BlockSpec((E_BLK, H), lambda i: (i, 0)),
            full((2 * H, 2 * H)),
            full((EA, 2 * H)),
            full((2 * H, H)),
            full((1, 2 * H)),
            full((1, H)),
            full((1, H)),
            full((1, H)),
        ],
        out_specs=pl.BlockSpec((E_BLK, H), lambda i: (i, 0)),
        out_shape=jax.ShapeDtypeStruct((N_EDGES, H), jnp.float32),
        compiler_params=pltpu.CompilerParams(
            dimension_semantics=("arbitrary",)),
    )(hsd, ea, eh, w1s, ct, w2t, b1, b2, g, bb)


N_BLK = 1000


def _node_body(nh_ref, a0_ref, a1_ref, dt_ref, et_ref, w2t_ref, b1_ref,
               b2_ref, g_ref, bb_ref, out_ref):
    agg = (a0_ref[...] + a1_ref[...]).astype(jnp.bfloat16)
    nh_bf = nh_ref[...].astype(jnp.bfloat16)
    x = (jnp.dot(nh_bf, dt_ref[...], preferred_element_type=jnp.float32)
         + jnp.dot(agg, et_ref[...], preferred_element_type=jnp.float32)
         + b1_ref[...])
    h = jnp.maximum(x, 0.0)
    upd = jnp.dot(h, w2t_ref[...], preferred_element_type=jnp.float32) + b2_ref[...]
    y = nh_ref[...] + upd
    mu = jnp.mean(y, axis=-1, keepdims=True)
    var = jnp.mean((y - mu) ** 2, axis=-1, keepdims=True)
    out_ref[...] = (y - mu) / jnp.sqrt(var + 1e-5) * g_ref[...] + bb_ref[...]


def _node_mlp(nh, a0, a1, dt, et, w2t, b1, b2, g, bb):
    grid = (N_NODES // N_BLK,)
    full = lambda shape: pl.BlockSpec(shape, lambda i: (0, 0))
    return pl.pallas_call(
        _node_body,
        grid=grid,
        in_specs=[
            pl.BlockSpec((N_BLK, H), lambda i: (i, 0)),
            pl.BlockSpec((N_BLK, H), lambda i: (i, 0)),
            pl.BlockSpec((N_BLK, H), lambda i: (i, 0)),
            full((H, 2 * H)),
            full((H, 2 * H)),
            full((2 * H, H)),
            full((1, 2 * H)),
            full((1, H)),
            full((1, H)),
            full((1, H)),
        ],
        out_specs=pl.BlockSpec((N_BLK, H), lambda i: (i, 0)),
        out_shape=jax.ShapeDtypeStruct((N_NODES, H), jnp.float32),
        compiler_params=pltpu.CompilerParams(
            dimension_semantics=("arbitrary",)),
    )(nh, a0, a1, dt, et, w2t, b1, b2, g, bb)


def kernel(node_h, edge_h, edge_index, edge_attr,
           W_e1, b_e1, W_e2, b_e2, W_n1, b_n1, W_n2, b_n2,
           ln_e_g, ln_e_b, ln_n_g, ln_n_b):
    ei = edge_index.astype(jnp.int32)
    src3 = ei[0].reshape(NW, NCH, CH)
    dst3 = ei[1].reshape(NW, NCH, CH)

    node_pack = lax.bitcast_convert_type(
        node_h.astype(jnp.bfloat16).reshape(N_NODES, HP, 2), jnp.int32)
    hsd = _sc_gather(node_pack, src3, dst3)

    at = W_e1[:, :H].T
    bt = W_e1[:, H:2 * H].T
    w1s = jnp.concatenate(
        [at[0::2], at[1::2], bt[0::2], bt[1::2]], axis=0).astype(jnp.bfloat16)
    ct = W_e1[:, 2 * H:].T.astype(jnp.bfloat16)
    w2t = W_e2.T.astype(jnp.bfloat16)
    ea_bf = edge_attr.astype(jnp.bfloat16)
    edge_h_new = _edge_mlp(hsd, ea_bf, edge_h, w1s, ct, w2t,
                           b_e1.reshape(1, -1), b_e2.reshape(1, -1),
                           ln_e_g.reshape(1, -1), ln_e_b.reshape(1, -1))

    zeros_pad = jnp.zeros((N_PAD, H), jnp.float32)
    parts = _sc_scatter(edge_h_new, dst3, zeros_pad)
    p0 = parts[0, :N_NODES]
    p1 = parts[1, :N_NODES]

    dt = W_n1[:, :H].T.astype(jnp.bfloat16)
    et = W_n1[:, H:].T.astype(jnp.bfloat16)
    wn2t = W_n2.T.astype(jnp.bfloat16)
    node_h_new = _node_mlp(node_h, p0, p1, dt, et, wn2t,
                           b_n1.reshape(1, -1), b_n2.reshape(1, -1),
                           ln_n_g.reshape(1, -1), ln_n_b.reshape(1, -1))
    return (node_h_new, edge_h_new)


# speedup vs baseline: 1.9091x; 1.1153x over previous
import functools

import jax
import jax.numpy as jnp
from jax import lax
from jax.experimental import pallas as pl
from jax.experimental.pallas import tpu as pltpu
from jax.experimental.pallas import tpu_sc as plsc

N_NODES = 10000
N_EDGES = 320000
H = 128
EA = 16

_INFO = plsc.get_sparse_core_info()
NC = _INFO.num_cores
NS = _INFO.num_subcores
NW = NC * NS
EPW = N_EDGES // NW
CH = 80
NCH = EPW // CH
N_PAD = 10240
ROWS_PER_TILE = N_PAD // NS

_mesh = plsc.VectorSubcoreMesh(core_axis_name="c", subcore_axis_name="s")


HP = H // 2


@functools.partial(
    pl.kernel,
    out_type=jax.ShapeDtypeStruct((N_EDGES, H), jnp.int32),
    mesh=_mesh,
    scratch_types=[
        pltpu.VMEM((NCH, CH), jnp.int32),
        pltpu.VMEM((NCH, CH), jnp.int32),
        pltpu.VMEM((CH, HP), jnp.int32),
        pltpu.VMEM((CH, HP), jnp.int32),
        pltpu.VMEM((CH, HP), jnp.int32),
        pltpu.VMEM((CH, HP), jnp.int32),
        pltpu.SemaphoreType.DMA,
        pltpu.SemaphoreType.DMA,
        pltpu.SemaphoreType.DMA,
        pltpu.SemaphoreType.DMA,
        pltpu.SemaphoreType.DMA,
        pltpu.SemaphoreType.DMA,
        pltpu.SemaphoreType.DMA,
        pltpu.SemaphoreType.DMA,
    ],
    compiler_params=pltpu.CompilerParams(use_tc_tiling_on_sc=False),
)
def _sc_gather(node_hbm, src3_hbm, dst3_hbm, hsd_hbm,
               idxs_v, idxd_v, rs0, rs1, rd0, rd1,
               gs0, gs1, gd0, gd1, ws0, ws1, wd0, wd1):
    c = lax.axis_index("c")
    s = lax.axis_index("s")
    wid = s * NC + c
    base_ch = wid * NCH
    pltpu.sync_copy(src3_hbm.at[wid], idxs_v)
    pltpu.sync_copy(dst3_hbm.at[wid], idxd_v)

    rs = (rs0, rs1)
    rd = (rd0, rd1)
    gs = (gs0, gs1)
    gd = (gd0, gd1)
    ws = (ws0, ws1)
    wd = (wd0, wd1)

    def wb_s(k, b):
        return pltpu.make_async_copy(
            rs[b], hsd_hbm.at[pl.ds((base_ch + k) * CH, CH), pl.ds(0, HP)],
            ws[b])

    def wb_d(k, b):
        return pltpu.make_async_copy(
            rd[b], hsd_hbm.at[pl.ds((base_ch + k) * CH, CH), pl.ds(HP, HP)],
            wd[b])

    pltpu.async_copy(node_hbm.at[idxs_v.at[0]], rs[0], gs[0])
    pltpu.async_copy(node_hbm.at[idxd_v.at[0]], rd[0], gd[0])

    @pl.loop(0, NCH - 1, step=2)
    def _pipe(j):
        for b in range(2):
            k = j + b
            pltpu.make_async_copy(node_hbm.at[idxs_v.at[k]], rs[b], gs[b]).wait()
            pltpu.make_async_copy(node_hbm.at[idxd_v.at[k]], rd[b], gd[b]).wait()
            if b == 1:
                wb_s(k - 1, 0).wait()
                wb_d(k - 1, 0).wait()
            else:
                @pl.when(j >= 1)
                def _():
                    wb_s(k - 1, 1).wait()
                    wb_d(k - 1, 1).wait()
            pltpu.async_copy(node_hbm.at[idxs_v.at[k + 1]], rs[1 - b], gs[1 - b])
            pltpu.async_copy(node_hbm.at[idxd_v.at[k + 1]], rd[1 - b], gd[1 - b])
            wb_s(k, b).start()
            wb_d(k, b).start()

    last = NCH - 1
    pltpu.make_async_copy(node_hbm.at[idxs_v.at[last]], rs[0], gs[0]).wait()
    pltpu.make_async_copy(node_hbm.at[idxd_v.at[last]], rd[0], gd[0]).wait()
    wb_s(last - 1, 1).wait()
    wb_d(last - 1, 1).wait()
    wb_s(last, 0).start()
    wb_d(last, 0).start()
    wb_s(last, 0).wait()
    wb_d(last, 0).wait()


@functools.partial(
    pl.kernel,
    out_type=jax.ShapeDtypeStruct((NC, N_PAD, H), jnp.float32),
    mesh=_mesh,
    scratch_types=[
        pltpu.VMEM((NCH, CH), jnp.int32),
        pltpu.VMEM((CH, H), jnp.float32),
        pltpu.VMEM((CH, H), jnp.float32),
        pltpu.VMEM_SHARED((N_PAD, H), jnp.float32),
        pltpu.SemaphoreType.DMA,
        pltpu.SemaphoreType.DMA,
    ],
)
def _sc_scatter(ehn_hbm, dst3_hbm, zeros_hbm, out_hbm,
                idx_v, r0, r1, agg_sh, rs0, rs1):
    c = lax.axis_index("c")
    s = lax.axis_index("s")
    wid = s * NC + c
    base_ch = wid * NCH

    row0 = s * ROWS_PER_TILE
    pltpu.sync_copy(zeros_hbm.at[pl.ds(row0, ROWS_PER_TILE)],
                    agg_sh.at[pl.ds(row0, ROWS_PER_TILE)])
    plsc.subcore_barrier()

    pltpu.sync_copy(dst3_hbm.at[wid], idx_v)

    rr = (r0, r1)
    ss = (rs0, rs1)

    def rd(k, b):
        return pltpu.make_async_copy(
            ehn_hbm.at[pl.ds((base_ch + k) * CH, CH)], rr[b], ss[b])

    rd(0, 0).start()

    @pl.loop(0, NCH - 1, step=2)
    def _pipe(j):
        for b in range(2):
            k = j + b
            rd(k, b).wait()
            rd(k + 1, 1 - b).start()
            pltpu.sync_copy(rr[b], agg_sh.at[idx_v.at[k]], add=True)

    last = NCH - 1
    rd(last, 0).wait()
    pltpu.sync_copy(rr[0], agg_sh.at[idx_v.at[last]], add=True)
    plsc.subcore_barrier()


    pltpu.sync_copy(agg_sh.at[pl.ds(row0, ROWS_PER_TILE)],
                    out_hbm.at[c, pl.ds(row0, ROWS_PER_TILE)])


E_BLK = 1280


def _unpack_bf16(p):
    even = lax.bitcast_convert_type(p << 16, jnp.float32).astype(jnp.bfloat16)
    odd = lax.bitcast_convert_type(p & jnp.int32(-65536),
                                   jnp.float32).astype(jnp.bfloat16)
    return even, odd


def _edge_body(hsd_ref, ea_ref, eh_ref, w1s_ref, ct_ref, w2t_ref,
               b1_ref, b2_ref, g_ref, bb_ref, out_ref):
    hsd = hsd_ref[...]
    hse, hso = _unpack_bf16(hsd[:, :HP])
    hde, hdo = _unpack_bf16(hsd[:, HP:])
    hcat = jnp.concatenate([hse, hso, hde, hdo], axis=1)
    x = (jnp.dot(hcat, w1s_ref[...], preferred_element_type=jnp.float32)
         + jnp.dot(ea_ref[...], ct_ref[...], preferred_element_type=jnp.float32)
         + b1_ref[...])
    h = jnp.maximum(x, 0.0)
    h_bf = h.astype(jnp.bfloat16)
    msg = jnp.dot(h_bf, w2t_ref[...],
                  preferred_element_type=jnp.float32) + b2_ref[...]
    y = eh_ref[...] + msg
    mu = jnp.mean(y, axis=-1, keepdims=True)
    var = jnp.mean((y - mu) ** 2, axis=-1, keepdims=True)
    out_ref[...] = (y - mu) / jnp.sqrt(var + 1e-5) * g_ref[...] + bb_ref[...]


def _edge_mlp(hsd, ea, eh, w1s, ct, w2t, b1, b2, g, bb):
    grid = (N_EDGES // E_BLK,)
    full = lambda shape: pl.BlockSpec(shape, lambda i: (0, 0))
    return pl.pallas_call(
        _edge_body,
        grid=grid,
        in_specs=[
            pl.BlockSpec((E_BLK, H), lambda i: (i, 0)),
            pl.BlockSpec((E_BLK, EA), lambda i: (i, 0)),
            pl.BlockSpec((E_BLK, H), lambda i: (i, 0)),
            full((2 * H, 2 * H)),
            full((EA, 2 * H)),
            full((2 * H, H)),
            full((1, 2 * H)),
            full((1, H)),
            full((1, H)),
            full((1, H)),
        ],
        out_specs=pl.BlockSpec((E_BLK, H), lambda i: (i, 0)),
        out_shape=jax.ShapeDtypeStruct((N_EDGES, H), jnp.float32),
        compiler_params=pltpu.CompilerParams(
            dimension_semantics=("arbitrary",)),
    )(hsd, ea, eh, w1s, ct, w2t, b1, b2, g, bb)


N_BLK = 1000


def _node_body(nh_ref, a0_ref, a1_ref, dt_ref, et_ref, w2t_ref, b1_ref,
               b2_ref, g_ref, bb_ref, out_ref):
    agg = (a0_ref[...] + a1_ref[...]).astype(jnp.bfloat16)
    nh_bf = nh_ref[...].astype(jnp.bfloat16)
    x = (jnp.dot(nh_bf, dt_ref[...], preferred_element_type=jnp.float32)
         + jnp.dot(agg, et_ref[...], preferred_element_type=jnp.float32)
         + b1_ref[...])
    h = jnp.maximum(x, 0.0)
    upd = jnp.dot(h, w2t_ref[...], preferred_element_type=jnp.float32) + b2_ref[...]
    y = nh_ref[...] + upd
    mu = jnp.mean(y, axis=-1, keepdims=True)
    var = jnp.mean((y - mu) ** 2, axis=-1, keepdims=True)
    out_ref[...] = (y - mu) / jnp.sqrt(var + 1e-5) * g_ref[...] + bb_ref[...]


def _node_mlp(nh, a0, a1, dt, et, w2t, b1, b2, g, bb):
    grid = (N_NODES // N_BLK,)
    full = lambda shape: pl.BlockSpec(shape, lambda i: (0, 0))
    return pl.pallas_call(
        _node_body,
        grid=grid,
        in_specs=[
            pl.BlockSpec((N_BLK, H), lambda i: (i, 0)),
            pl.BlockSpec((N_BLK, H), lambda i: (i, 0)),
            pl.BlockSpec((N_BLK, H), lambda i: (i, 0)),
            full((H, 2 * H)),
            full((H, 2 * H)),
            full((2 * H, H)),
            full((1, 2 * H)),
            full((1, H)),
            full((1, H)),
            full((1, H)),
        ],
        out_specs=pl.BlockSpec((N_BLK, H), lambda i: (i, 0)),
        out_shape=jax.ShapeDtypeStruct((N_NODES, H), jnp.float32),
        compiler_params=pltpu.CompilerParams(
            dimension_semantics=("arbitrary",)),
    )(nh, a0, a1, dt, et, w2t, b1, b2, g, bb)


def kernel(node_h, edge_h, edge_index, edge_attr,
           W_e1, b_e1, W_e2, b_e2, W_n1, b_n1, W_n2, b_n2,
           ln_e_g, ln_e_b, ln_n_g, ln_n_b):
    ei = edge_index.astype(jnp.int32)
    src3 = ei[0].reshape(NW, NCH, CH)
    dst3 = ei[1].reshape(NW, NCH, CH)

    node_pack = lax.bitcast_convert_type(
        node_h.astype(jnp.bfloat16).reshape(N_NODES, HP, 2), jnp.int32)
    hsd = _sc_gather(node_pack, src3, dst3)

    at = W_e1[:, :H].T
    bt = W_e1[:, H:2 * H].T
    w1s = jnp.concatenate(
        [at[0::2], at[1::2], bt[0::2], bt[1::2]], axis=0).astype(jnp.bfloat16)
    ct = W_e1[:, 2 * H:].T.astype(jnp.bfloat16)
    w2t = W_e2.T.astype(jnp.bfloat16)
    ea_bf = edge_attr.astype(jnp.bfloat16)
    edge_h_new = _edge_mlp(hsd, ea_bf, edge_h, w1s, ct, w2t,
                           b_e1.reshape(1, -1), b_e2.reshape(1, -1),
                           ln_e_g.reshape(1, -1), ln_e_b.reshape(1, -1))

    zeros_pad = jnp.zeros((N_PAD, H), jnp.float32)
    parts = _sc_scatter(edge_h_new, dst3, zeros_pad)
    p0 = parts[0, :N_NODES]
    p1 = parts[1, :N_NODES]

    dt = W_n1[:, :H].T.astype(jnp.bfloat16)
    et = W_n1[:, H:].T.astype(jnp.bfloat16)
    wn2t = W_n2.T.astype(jnp.bfloat16)
    node_h_new = _node_mlp(node_h, p0, p1, dt, et, wn2t,
                           b_n1.reshape(1, -1), b_n2.reshape(1, -1),
                           ln_n_g.reshape(1, -1), ln_n_b.reshape(1, -1))
    return (node_h_new, edge_h_new)


# speedup vs baseline: 2.1390x; 1.1204x over previous
import functools

import jax
import jax.numpy as jnp
from jax import lax
from jax.experimental import pallas as pl
from jax.experimental.pallas import tpu as pltpu
from jax.experimental.pallas import tpu_sc as plsc

N_NODES = 10000
N_EDGES = 320000
H = 128
EA = 16

_INFO = plsc.get_sparse_core_info()
NC = _INFO.num_cores
NS = _INFO.num_subcores
NW = NC * NS
EPW = N_EDGES // NW
CH = 80
NCH = EPW // CH
N_PAD = 10240
ROWS_PER_TILE = N_PAD // NS

_mesh = plsc.VectorSubcoreMesh(core_axis_name="c", subcore_axis_name="s")


HP = H // 2


@functools.partial(
    pl.kernel,
    out_type=jax.ShapeDtypeStruct((N_EDGES, H), jnp.int32),
    mesh=_mesh,
    scratch_types=[
        pltpu.VMEM((NCH, CH), jnp.int32),
        pltpu.VMEM((NCH, CH), jnp.int32),
        pltpu.VMEM((CH, HP), jnp.int32),
        pltpu.VMEM((CH, HP), jnp.int32),
        pltpu.VMEM((CH, HP), jnp.int32),
        pltpu.VMEM((CH, HP), jnp.int32),
        pltpu.SemaphoreType.DMA,
        pltpu.SemaphoreType.DMA,
        pltpu.SemaphoreType.DMA,
        pltpu.SemaphoreType.DMA,
        pltpu.SemaphoreType.DMA,
        pltpu.SemaphoreType.DMA,
        pltpu.SemaphoreType.DMA,
        pltpu.SemaphoreType.DMA,
    ],
    compiler_params=pltpu.CompilerParams(use_tc_tiling_on_sc=False),
)
def _sc_gather(node_hbm, src3_hbm, dst3_hbm, hsd_hbm,
               idxs_v, idxd_v, rs0, rs1, rd0, rd1,
               gs0, gs1, gd0, gd1, ws0, ws1, wd0, wd1):
    c = lax.axis_index("c")
    s = lax.axis_index("s")
    wid = s * NC + c
    base_ch = wid * NCH
    pltpu.sync_copy(src3_hbm.at[wid], idxs_v)
    pltpu.sync_copy(dst3_hbm.at[wid], idxd_v)

    rs = (rs0, rs1)
    rd = (rd0, rd1)
    gs = (gs0, gs1)
    gd = (gd0, gd1)
    ws = (ws0, ws1)
    wd = (wd0, wd1)

    def wb_s(k, b):
        return pltpu.make_async_copy(
            rs[b], hsd_hbm.at[pl.ds((base_ch + k) * CH, CH), pl.ds(0, HP)],
            ws[b])

    def wb_d(k, b):
        return pltpu.make_async_copy(
            rd[b], hsd_hbm.at[pl.ds((base_ch + k) * CH, CH), pl.ds(HP, HP)],
            wd[b])

    pltpu.async_copy(node_hbm.at[idxs_v.at[0]], rs[0], gs[0])
    pltpu.async_copy(node_hbm.at[idxd_v.at[0]], rd[0], gd[0])

    @pl.loop(0, NCH - 1, step=2)
    def _pipe(j):
        for b in range(2):
            k = j + b
            pltpu.make_async_copy(node_hbm.at[idxs_v.at[k]], rs[b], gs[b]).wait()
            pltpu.make_async_copy(node_hbm.at[idxd_v.at[k]], rd[b], gd[b]).wait()
            if b == 1:
                wb_s(k - 1, 0).wait()
                wb_d(k - 1, 0).wait()
            else:
                @pl.when(j >= 1)
                def _():
                    wb_s(k - 1, 1).wait()
                    wb_d(k - 1, 1).wait()
            pltpu.async_copy(node_hbm.at[idxs_v.at[k + 1]], rs[1 - b], gs[1 - b])
            pltpu.async_copy(node_hbm.at[idxd_v.at[k + 1]], rd[1 - b], gd[1 - b])
            wb_s(k, b).start()
            wb_d(k, b).start()

    last = NCH - 1
    pltpu.make_async_copy(node_hbm.at[idxs_v.at[last]], rs[0], gs[0]).wait()
    pltpu.make_async_copy(node_hbm.at[idxd_v.at[last]], rd[0], gd[0]).wait()
    wb_s(last - 1, 1).wait()
    wb_d(last - 1, 1).wait()
    wb_s(last, 0).start()
    wb_d(last, 0).start()
    wb_s(last, 0).wait()
    wb_d(last, 0).wait()


@functools.partial(
    pl.kernel,
    out_type=jax.ShapeDtypeStruct((NC, N_PAD, H), jnp.float32),
    mesh=_mesh,
    scratch_types=[
        pltpu.VMEM((NCH, CH), jnp.int32),
        pltpu.VMEM((CH, H), jnp.float32),
        pltpu.VMEM((CH, H), jnp.float32),
        pltpu.VMEM_SHARED((N_PAD, H), jnp.float32),
        pltpu.SemaphoreType.DMA,
        pltpu.SemaphoreType.DMA,
    ],
)
def _sc_scatter(ehn_hbm, dst3_hbm, zeros_hbm, out_hbm,
                idx_v, r0, r1, agg_sh, rs0, rs1):
    c = lax.axis_index("c")
    s = lax.axis_index("s")
    wid = s * NC + c
    base_ch = wid * NCH

    row0 = s * ROWS_PER_TILE
    pltpu.sync_copy(zeros_hbm.at[pl.ds(row0, ROWS_PER_TILE)],
                    agg_sh.at[pl.ds(row0, ROWS_PER_TILE)])
    plsc.subcore_barrier()

    pltpu.sync_copy(dst3_hbm.at[wid], idx_v)

    rr = (r0, r1)
    ss = (rs0, rs1)

    def rd(k, b):
        return pltpu.make_async_copy(
            ehn_hbm.at[pl.ds((base_ch + k) * CH, CH)], rr[b], ss[b])

    rd(0, 0).start()

    @pl.loop(0, NCH - 1, step=2)
    def _pipe(j):
        for b in range(2):
            k = j + b
            rd(k, b).wait()
            rd(k + 1, 1 - b).start()
            pltpu.sync_copy(rr[b], agg_sh.at[idx_v.at[k]], add=True)

    last = NCH - 1
    rd(last, 0).wait()
    pltpu.sync_copy(rr[0], agg_sh.at[idx_v.at[last]], add=True)
    plsc.subcore_barrier()


    pltpu.sync_copy(agg_sh.at[pl.ds(row0, ROWS_PER_TILE)],
                    out_hbm.at[c, pl.ds(row0, ROWS_PER_TILE)])


E_BLK = 2560


def _unpack_bf16(p):
    even = lax.bitcast_convert_type(p << 16, jnp.float32).astype(jnp.bfloat16)
    odd = lax.bitcast_convert_type(p & jnp.int32(-65536),
                                   jnp.float32).astype(jnp.bfloat16)
    return even, odd


def _edge_body(hsd_ref, ea_ref, eh_ref, w1s_ref, ct_ref, w2t_ref,
               b1_ref, b2_ref, g_ref, bb_ref, out_ref):
    hsd = hsd_ref[...]
    hse, hso = _unpack_bf16(hsd[:, :HP])
    hde, hdo = _unpack_bf16(hsd[:, HP:])
    hcat = jnp.concatenate([hse, hso, hde, hdo], axis=1)
    x = (jnp.dot(hcat, w1s_ref[...], preferred_element_type=jnp.float32)
         + jnp.dot(ea_ref[...], ct_ref[...], preferred_element_type=jnp.float32)
         + b1_ref[...])
    h = jnp.maximum(x, 0.0)
    h_bf = h.astype(jnp.bfloat16)
    msg = jnp.dot(h_bf, w2t_ref[...],
                  preferred_element_type=jnp.float32) + b2_ref[...]
    y = eh_ref[...] + msg
    mu = jnp.mean(y, axis=-1, keepdims=True)
    var = jnp.mean((y - mu) ** 2, axis=-1, keepdims=True)
    out_ref[...] = (y - mu) / jnp.sqrt(var + 1e-5) * g_ref[...] + bb_ref[...]


def _edge_mlp(hsd, ea, eh, w1s, ct, w2t, b1, b2, g, bb):
    grid = (N_EDGES // E_BLK,)
    full = lambda shape: pl.BlockSpec(shape, lambda i: (0, 0))
    return pl.pallas_call(
        _edge_body,
        grid=grid,
        in_specs=[
            pl.BlockSpec((E_BLK, H), lambda i: (i, 0)),
            pl.BlockSpec((E_BLK, EA), lambda i: (i, 0)),
            pl.BlockSpec((E_BLK, H), lambda i: (i, 0)),
            full((2 * H, 2 * H)),
            full((EA, 2 * H)),
            full((2 * H, H)),
            full((1, 2 * H)),
            full((1, H)),
            full((1, H)),
            full((1, H)),
        ],
        out_specs=pl.BlockSpec((E_BLK, H), lambda i: (i, 0)),
        out_shape=jax.ShapeDtypeStruct((N_EDGES, H), jnp.float32),
        compiler_params=pltpu.CompilerParams(
            dimension_semantics=("arbitrary",)),
    )(hsd, ea, eh, w1s, ct, w2t, b1, b2, g, bb)


N_BLK = 1000


def _node_body(nh_ref, a0_ref, a1_ref, dt_ref, et_ref, w2t_ref, b1_ref,
               b2_ref, g_ref, bb_ref, out_ref):
    agg = (a0_ref[...] + a1_ref[...]).astype(jnp.bfloat16)
    nh_bf = nh_ref[...].astype(jnp.bfloat16)
    x = (jnp.dot(nh_bf, dt_ref[...], preferred_element_type=jnp.float32)
         + jnp.dot(agg, et_ref[...], preferred_element_type=jnp.float32)
         + b1_ref[...])
    h = jnp.maximum(x, 0.0)
    upd = jnp.dot(h, w2t_ref[...], preferred_element_type=jnp.float32) + b2_ref[...]
    y = nh_ref[...] + upd
    mu = jnp.mean(y, axis=-1, keepdims=True)
    var = jnp.mean((y - mu) ** 2, axis=-1, keepdims=True)
    out_ref[...] = (y - mu) / jnp.sqrt(var + 1e-5) * g_ref[...] + bb_ref[...]


def _node_mlp(nh, a0, a1, dt, et, w2t, b1, b2, g, bb):
    grid = (N_NODES // N_BLK,)
    full = lambda shape: pl.BlockSpec(shape, lambda i: (0, 0))
    return pl.pallas_call(
        _node_body,
        grid=grid,
        in_specs=[
            pl.BlockSpec((N_BLK, H), lambda i: (i, 0)),
            pl.BlockSpec((N_BLK, H), lambda i: (i, 0)),
            pl.BlockSpec((N_BLK, H), lambda i: (i, 0)),
            full((H, 2 * H)),
            full((H, 2 * H)),
            full((2 * H, H)),
            full((1, 2 * H)),
            full((1, H)),
            full((1, H)),
            full((1, H)),
        ],
        out_specs=pl.BlockSpec((N_BLK, H), lambda i: (i, 0)),
        out_shape=jax.ShapeDtypeStruct((N_NODES, H), jnp.float32),
        compiler_params=pltpu.CompilerParams(
            dimension_semantics=("arbitrary",)),
    )(nh, a0, a1, dt, et, w2t, b1, b2, g, bb)


def kernel(node_h, edge_h, edge_index, edge_attr,
           W_e1, b_e1, W_e2, b_e2, W_n1, b_n1, W_n2, b_n2,
           ln_e_g, ln_e_b, ln_n_g, ln_n_b):
    ei = edge_index.astype(jnp.int32)
    src3 = ei[0].reshape(NW, NCH, CH)
    dst3 = ei[1].reshape(NW, NCH, CH)

    node_pack = lax.bitcast_convert_type(
        node_h.astype(jnp.bfloat16).reshape(N_NODES, HP, 2), jnp.int32)
    hsd = _sc_gather(node_pack, src3, dst3)

    at = W_e1[:, :H].T
    bt = W_e1[:, H:2 * H].T
    w1s = jnp.concatenate(
        [at[0::2], at[1::2], bt[0::2], bt[1::2]], axis=0).astype(jnp.bfloat16)
    ct = W_e1[:, 2 * H:].T.astype(jnp.bfloat16)
    w2t = W_e2.T.astype(jnp.bfloat16)
    ea_bf = edge_attr.astype(jnp.bfloat16)
    edge_h_new = _edge_mlp(hsd, ea_bf, edge_h, w1s, ct, w2t,
                           b_e1.reshape(1, -1), b_e2.reshape(1, -1),
                           ln_e_g.reshape(1, -1), ln_e_b.reshape(1, -1))

    zeros_pad = jnp.zeros((N_PAD, H), jnp.float32)
    parts = _sc_scatter(edge_h_new, dst3, zeros_pad)
    p0 = parts[0, :N_NODES]
    p1 = parts[1, :N_NODES]

    dt = W_n1[:, :H].T.astype(jnp.bfloat16)
    et = W_n1[:, H:].T.astype(jnp.bfloat16)
    wn2t = W_n2.T.astype(jnp.bfloat16)
    node_h_new = _node_mlp(node_h, p0, p1, dt, et, wn2t,
                           b_n1.reshape(1, -1), b_n2.reshape(1, -1),
                           ln_n_g.reshape(1, -1), ln_n_b.reshape(1, -1))
    return (node_h_new, edge_h_new)


# speedup vs baseline: 2.2347x; 1.0448x over previous
import functools

import jax
import jax.numpy as jnp
from jax import lax
from jax.experimental import pallas as pl
from jax.experimental.pallas import tpu as pltpu
from jax.experimental.pallas import tpu_sc as plsc

N_NODES = 10000
N_EDGES = 320000
H = 128
EA = 16

_INFO = plsc.get_sparse_core_info()
NC = _INFO.num_cores
NS = _INFO.num_subcores
NW = NC * NS
EPW = N_EDGES // NW
CH = 80
NCH = EPW // CH
N_PAD = 10240
ROWS_PER_TILE = N_PAD // NS

_mesh = plsc.VectorSubcoreMesh(core_axis_name="c", subcore_axis_name="s")


HP = H // 2


@functools.partial(
    pl.kernel,
    out_type=jax.ShapeDtypeStruct((N_EDGES, H), jnp.int32),
    mesh=_mesh,
    scratch_types=[
        pltpu.VMEM((NCH, CH), jnp.int32),
        pltpu.VMEM((NCH, CH), jnp.int32),
        pltpu.VMEM((CH, HP), jnp.int32),
        pltpu.VMEM((CH, HP), jnp.int32),
        pltpu.VMEM((CH, HP), jnp.int32),
        pltpu.VMEM((CH, HP), jnp.int32),
        pltpu.SemaphoreType.DMA,
        pltpu.SemaphoreType.DMA,
        pltpu.SemaphoreType.DMA,
        pltpu.SemaphoreType.DMA,
        pltpu.SemaphoreType.DMA,
        pltpu.SemaphoreType.DMA,
        pltpu.SemaphoreType.DMA,
        pltpu.SemaphoreType.DMA,
    ],
    compiler_params=pltpu.CompilerParams(use_tc_tiling_on_sc=False),
)
def _sc_gather(node_hbm, src3_hbm, dst3_hbm, hsd_hbm,
               idxs_v, idxd_v, rs0, rs1, rd0, rd1,
               gs0, gs1, gd0, gd1, ws0, ws1, wd0, wd1):
    c = lax.axis_index("c")
    s = lax.axis_index("s")
    wid = s * NC + c
    base_ch = wid * NCH
    pltpu.sync_copy(src3_hbm.at[wid], idxs_v)
    pltpu.sync_copy(dst3_hbm.at[wid], idxd_v)

    rs = (rs0, rs1)
    rd = (rd0, rd1)
    gs = (gs0, gs1)
    gd = (gd0, gd1)
    ws = (ws0, ws1)
    wd = (wd0, wd1)

    def wb_s(k, b):
        return pltpu.make_async_copy(
            rs[b], hsd_hbm.at[pl.ds((base_ch + k) * CH, CH), pl.ds(0, HP)],
            ws[b])

    def wb_d(k, b):
        return pltpu.make_async_copy(
            rd[b], hsd_hbm.at[pl.ds((base_ch + k) * CH, CH), pl.ds(HP, HP)],
            wd[b])

    pltpu.async_copy(node_hbm.at[idxs_v.at[0]], rs[0], gs[0])
    pltpu.async_copy(node_hbm.at[idxd_v.at[0]], rd[0], gd[0])

    @pl.loop(0, NCH - 1, step=2)
    def _pipe(j):
        for b in range(2):
            k = j + b
            pltpu.make_async_copy(node_hbm.at[idxs_v.at[k]], rs[b], gs[b]).wait()
            pltpu.make_async_copy(node_hbm.at[idxd_v.at[k]], rd[b], gd[b]).wait()
            if b == 1:
                wb_s(k - 1, 0).wait()
                wb_d(k - 1, 0).wait()
            else:
                @pl.when(j >= 1)
                def _():
                    wb_s(k - 1, 1).wait()
                    wb_d(k - 1, 1).wait()
            pltpu.async_copy(node_hbm.at[idxs_v.at[k + 1]], rs[1 - b], gs[1 - b])
            pltpu.async_copy(node_hbm.at[idxd_v.at[k + 1]], rd[1 - b], gd[1 - b])
            wb_s(k, b).start()
            wb_d(k, b).start()

    last = NCH - 1
    pltpu.make_async_copy(node_hbm.at[idxs_v.at[last]], rs[0], gs[0]).wait()
    pltpu.make_async_copy(node_hbm.at[idxd_v.at[last]], rd[0], gd[0]).wait()
    wb_s(last - 1, 1).wait()
    wb_d(last - 1, 1).wait()
    wb_s(last, 0).start()
    wb_d(last, 0).start()
    wb_s(last, 0).wait()
    wb_d(last, 0).wait()


@functools.partial(
    pl.kernel,
    out_type=jax.ShapeDtypeStruct((NC, N_PAD, H), jnp.float32),
    mesh=_mesh,
    scratch_types=[
        pltpu.VMEM((NCH, CH), jnp.int32),
        pltpu.VMEM((CH, H), jnp.float32),
        pltpu.VMEM((CH, H), jnp.float32),
        pltpu.VMEM_SHARED((N_PAD, H), jnp.float32),
        pltpu.SemaphoreType.DMA,
        pltpu.SemaphoreType.DMA,
    ],
)
def _sc_scatter(ehn_hbm, dst3_hbm, zeros_hbm, out_hbm,
                idx_v, r0, r1, agg_sh, rs0, rs1):
    c = lax.axis_index("c")
    s = lax.axis_index("s")
    wid = s * NC + c
    base_ch = wid * NCH

    row0 = s * ROWS_PER_TILE
    pltpu.sync_copy(zeros_hbm.at[pl.ds(row0, ROWS_PER_TILE)],
                    agg_sh.at[pl.ds(row0, ROWS_PER_TILE)])
    plsc.subcore_barrier()

    pltpu.sync_copy(dst3_hbm.at[wid], idx_v)

    rr = (r0, r1)
    ss = (rs0, rs1)

    def rd(k, b):
        return pltpu.make_async_copy(
            ehn_hbm.at[pl.ds((base_ch + k) * CH, CH)], rr[b], ss[b])

    rd(0, 0).start()

    @pl.loop(0, NCH - 1, step=2)
    def _pipe(j):
        for b in range(2):
            k = j + b
            rd(k, b).wait()
            rd(k + 1, 1 - b).start()
            pltpu.sync_copy(rr[b], agg_sh.at[idx_v.at[k]], add=True)

    last = NCH - 1
    rd(last, 0).wait()
    pltpu.sync_copy(rr[0], agg_sh.at[idx_v.at[last]], add=True)
    plsc.subcore_barrier()


    pltpu.sync_copy(agg_sh.at[pl.ds(row0, ROWS_PER_TILE)],
                    out_hbm.at[c, pl.ds(row0, ROWS_PER_TILE)])


E_BLK = 4000


def _unpack_bf16(p):
    even = lax.bitcast_convert_type(p << 16, jnp.float32).astype(jnp.bfloat16)
    odd = lax.bitcast_convert_type(p & jnp.int32(-65536),
                                   jnp.float32).astype(jnp.bfloat16)
    return even, odd


def _edge_body(hsd_ref, ea_ref, eh_ref, w1s_ref, ct_ref, w2t_ref,
               b1_ref, b2_ref, g_ref, bb_ref, out_ref):
    hsd = hsd_ref[...]
    hse, hso = _unpack_bf16(hsd[:, :HP])
    hde, hdo = _unpack_bf16(hsd[:, HP:])
    hcat = jnp.concatenate([hse, hso, hde, hdo], axis=1)
    x = (jnp.dot(hcat, w1s_ref[...], preferred_element_type=jnp.float32)
         + jnp.dot(ea_ref[...], ct_ref[...], preferred_element_type=jnp.float32)
         + b1_ref[...])
    h = jnp.maximum(x, 0.0)
    h_bf = h.astype(jnp.bfloat16)
    msg = jnp.dot(h_bf, w2t_ref[...],
                  preferred_element_type=jnp.float32) + b2_ref[...]
    y = eh_ref[...] + msg
    mu = jnp.mean(y, axis=-1, keepdims=True)
    var = jnp.mean((y - mu) ** 2, axis=-1, keepdims=True)
    out_ref[...] = (y - mu) / jnp.sqrt(var + 1e-5) * g_ref[...] + bb_ref[...]


def _edge_mlp(hsd, ea, eh, w1s, ct, w2t, b1, b2, g, bb):
    grid = (N_EDGES // E_BLK,)
    full = lambda shape: pl.BlockSpec(shape, lambda i: (0, 0))
    return pl.pallas_call(
        _edge_body,
        grid=grid,
        in_specs=[
            pl.BlockSpec((E_BLK, H), lambda i: (i, 0)),
            pl.BlockSpec((E_BLK, EA), lambda i: (i, 0)),
            pl.BlockSpec((E_BLK, H), lambda i: (i, 0)),
            full((2 * H, 2 * H)),
            full((EA, 2 * H)),
            full((2 * H, H)),
            full((1, 2 * H)),
            full((1, H)),
            full((1, H)),
            full((1, H)),
        ],
        out_specs=pl.BlockSpec((E_BLK, H), lambda i: (i, 0)),
        out_shape=jax.ShapeDtypeStruct((N_EDGES, H), jnp.float32),
        compiler_params=pltpu.CompilerParams(
            dimension_semantics=("arbitrary",)),
    )(hsd, ea, eh, w1s, ct, w2t, b1, b2, g, bb)


N_BLK = 1000


def _node_body(nh_ref, a0_ref, a1_ref, dt_ref, et_ref, w2t_ref, b1_ref,
               b2_ref, g_ref, bb_ref, out_ref):
    agg = (a0_ref[...] + a1_ref[...]).astype(jnp.bfloat16)
    nh_bf = nh_ref[...].astype(jnp.bfloat16)
    x = (jnp.dot(nh_bf, dt_ref[...], preferred_element_type=jnp.float32)
         + jnp.dot(agg, et_ref[...], preferred_element_type=jnp.float32)
         + b1_ref[...])
    h = jnp.maximum(x, 0.0)
    upd = jnp.dot(h, w2t_ref[...], preferred_element_type=jnp.float32) + b2_ref[...]
    y = nh_ref[...] + upd
    mu = jnp.mean(y, axis=-1, keepdims=True)
    var = jnp.mean((y - mu) ** 2, axis=-1, keepdims=True)
    out_ref[...] = (y - mu) / jnp.sqrt(var + 1e-5) * g_ref[...] + bb_ref[...]


def _node_mlp(nh, a0, a1, dt, et, w2t, b1, b2, g, bb):
    grid = (N_NODES // N_BLK,)
    full = lambda shape: pl.BlockSpec(shape, lambda i: (0, 0))
    return pl.pallas_call(
        _node_body,
        grid=grid,
        in_specs=[
            pl.BlockSpec((N_BLK, H), lambda i: (i, 0)),
            pl.BlockSpec((N_BLK, H), lambda i: (i, 0)),
            pl.BlockSpec((N_BLK, H), lambda i: (i, 0)),
            full((H, 2 * H)),
            full((H, 2 * H)),
            full((2 * H, H)),
            full((1, 2 * H)),
            full((1, H)),
            full((1, H)),
            full((1, H)),
        ],
        out_specs=pl.BlockSpec((N_BLK, H), lambda i: (i, 0)),
        out_shape=jax.ShapeDtypeStruct((N_NODES, H), jnp.float32),
        compiler_params=pltpu.CompilerParams(
            dimension_semantics=("arbitrary",)),
    )(nh, a0, a1, dt, et, w2t, b1, b2, g, bb)


def kernel(node_h, edge_h, edge_index, edge_attr,
           W_e1, b_e1, W_e2, b_e2, W_n1, b_n1, W_n2, b_n2,
           ln_e_g, ln_e_b, ln_n_g, ln_n_b):
    ei = edge_index.astype(jnp.int32)
    src3 = ei[0].reshape(NW, NCH, CH)
    dst3 = ei[1].reshape(NW, NCH, CH)

    node_pack = lax.bitcast_convert_type(
        node_h.astype(jnp.bfloat16).reshape(N_NODES, HP, 2), jnp.int32)
    hsd = _sc_gather(node_pack, src3, dst3)

    at = W_e1[:, :H].T
    bt = W_e1[:, H:2 * H].T
    w1s = jnp.concatenate(
        [at[0::2], at[1::2], bt[0::2], bt[1::2]], axis=0).astype(jnp.bfloat16)
    ct = W_e1[:, 2 * H:].T.astype(jnp.bfloat16)
    w2t = W_e2.T.astype(jnp.bfloat16)
    ea_bf = edge_attr.astype(jnp.bfloat16)
    edge_h_new = _edge_mlp(hsd, ea_bf, edge_h, w1s, ct, w2t,
                           b_e1.reshape(1, -1), b_e2.reshape(1, -1),
                           ln_e_g.reshape(1, -1), ln_e_b.reshape(1, -1))

    zeros_pad = jnp.zeros((N_PAD, H), jnp.float32)
    parts = _sc_scatter(edge_h_new, dst3, zeros_pad)
    p0 = parts[0, :N_NODES]
    p1 = parts[1, :N_NODES]

    dt = W_n1[:, :H].T.astype(jnp.bfloat16)
    et = W_n1[:, H:].T.astype(jnp.bfloat16)
    wn2t = W_n2.T.astype(jnp.bfloat16)
    node_h_new = _node_mlp(node_h, p0, p1, dt, et, wn2t,
                           b_n1.reshape(1, -1), b_n2.reshape(1, -1),
                           ln_n_g.reshape(1, -1), ln_n_b.reshape(1, -1))
    return (node_h_new, edge_h_new)


# speedup vs baseline: 2.3251x; 1.0405x over previous
import functools

import jax
import jax.numpy as jnp
from jax import lax
from jax.experimental import pallas as pl
from jax.experimental.pallas import tpu as pltpu
from jax.experimental.pallas import tpu_sc as plsc

N_NODES = 10000
N_EDGES = 320000
H = 128
EA = 16

_INFO = plsc.get_sparse_core_info()
NC = _INFO.num_cores
NS = _INFO.num_subcores
NW = NC * NS
EPW = N_EDGES // NW
CH = 80
NCH = EPW // CH
N_PAD = 10240
ROWS_PER_TILE = N_PAD // NS

_mesh = plsc.VectorSubcoreMesh(core_axis_name="c", subcore_axis_name="s")


HP = H // 2


@functools.partial(
    pl.kernel,
    out_type=jax.ShapeDtypeStruct((N_EDGES, H), jnp.int32),
    mesh=_mesh,
    scratch_types=[
        pltpu.VMEM((NCH, CH), jnp.int32),
        pltpu.VMEM((NCH, CH), jnp.int32),
        pltpu.VMEM((CH, HP), jnp.int32),
        pltpu.VMEM((CH, HP), jnp.int32),
        pltpu.VMEM((CH, HP), jnp.int32),
        pltpu.VMEM((CH, HP), jnp.int32),
        pltpu.SemaphoreType.DMA,
        pltpu.SemaphoreType.DMA,
        pltpu.SemaphoreType.DMA,
        pltpu.SemaphoreType.DMA,
        pltpu.SemaphoreType.DMA,
        pltpu.SemaphoreType.DMA,
        pltpu.SemaphoreType.DMA,
        pltpu.SemaphoreType.DMA,
    ],
    compiler_params=pltpu.CompilerParams(use_tc_tiling_on_sc=False),
)
def _sc_gather(node_hbm, src3_hbm, dst3_hbm, hsd_hbm,
               idxs_v, idxd_v, rs0, rs1, rd0, rd1,
               gs0, gs1, gd0, gd1, ws0, ws1, wd0, wd1):
    c = lax.axis_index("c")
    s = lax.axis_index("s")
    wid = s * NC + c
    base_ch = wid * NCH
    pltpu.sync_copy(src3_hbm.at[wid], idxs_v)
    pltpu.sync_copy(dst3_hbm.at[wid], idxd_v)

    rs = (rs0, rs1)
    rd = (rd0, rd1)
    gs = (gs0, gs1)
    gd = (gd0, gd1)
    ws = (ws0, ws1)
    wd = (wd0, wd1)

    def wb_s(k, b):
        return pltpu.make_async_copy(
            rs[b], hsd_hbm.at[pl.ds((base_ch + k) * CH, CH), pl.ds(0, HP)],
            ws[b])

    def wb_d(k, b):
        return pltpu.make_async_copy(
            rd[b], hsd_hbm.at[pl.ds((base_ch + k) * CH, CH), pl.ds(HP, HP)],
            wd[b])

    pltpu.async_copy(node_hbm.at[idxs_v.at[0]], rs[0], gs[0])
    pltpu.async_copy(node_hbm.at[idxd_v.at[0]], rd[0], gd[0])

    @pl.loop(0, NCH - 1, step=2)
    def _pipe(j):
        for b in range(2):
            k = j + b
            pltpu.make_async_copy(node_hbm.at[idxs_v.at[k]], rs[b], gs[b]).wait()
            pltpu.make_async_copy(node_hbm.at[idxd_v.at[k]], rd[b], gd[b]).wait()
            if b == 1:
                wb_s(k - 1, 0).wait()
                wb_d(k - 1, 0).wait()
            else:
                @pl.when(j >= 1)
                def _():
                    wb_s(k - 1, 1).wait()
                    wb_d(k - 1, 1).wait()
            pltpu.async_copy(node_hbm.at[idxs_v.at[k + 1]], rs[1 - b], gs[1 - b])
            pltpu.async_copy(node_hbm.at[idxd_v.at[k + 1]], rd[1 - b], gd[1 - b])
            wb_s(k, b).start()
            wb_d(k, b).start()

    last = NCH - 1
    pltpu.make_async_copy(node_hbm.at[idxs_v.at[last]], rs[0], gs[0]).wait()
    pltpu.make_async_copy(node_hbm.at[idxd_v.at[last]], rd[0], gd[0]).wait()
    wb_s(last - 1, 1).wait()
    wb_d(last - 1, 1).wait()
    wb_s(last, 0).start()
    wb_d(last, 0).start()
    wb_s(last, 0).wait()
    wb_d(last, 0).wait()


@functools.partial(
    pl.kernel,
    out_type=jax.ShapeDtypeStruct((NC, N_PAD, H), jnp.float32),
    mesh=_mesh,
    scratch_types=[
        pltpu.VMEM((NCH, CH), jnp.int32),
        pltpu.VMEM((CH, H), jnp.float32),
        pltpu.VMEM((CH, H), jnp.float32),
        pltpu.VMEM_SHARED((N_PAD, H), jnp.float32),
        pltpu.SemaphoreType.DMA,
        pltpu.SemaphoreType.DMA,
    ],
)
def _sc_scatter(ehn_hbm, dst3_hbm, zeros_hbm, out_hbm,
                idx_v, r0, r1, agg_sh, rs0, rs1):
    c = lax.axis_index("c")
    s = lax.axis_index("s")
    wid = s * NC + c
    base_ch = wid * NCH

    row0 = s * ROWS_PER_TILE
    pltpu.sync_copy(zeros_hbm.at[pl.ds(row0, ROWS_PER_TILE)],
                    agg_sh.at[pl.ds(row0, ROWS_PER_TILE)])
    plsc.subcore_barrier()

    pltpu.sync_copy(dst3_hbm.at[wid], idx_v)

    rr = (r0, r1)
    ss = (rs0, rs1)

    def rd(k, b):
        return pltpu.make_async_copy(
            ehn_hbm.at[pl.ds((base_ch + k) * CH, CH)], rr[b], ss[b])

    rd(0, 0).start()

    @pl.loop(0, NCH - 1, step=2)
    def _pipe(j):
        for b in range(2):
            k = j + b
            rd(k, b).wait()
            rd(k + 1, 1 - b).start()
            pltpu.sync_copy(rr[b], agg_sh.at[idx_v.at[k]], add=True)

    last = NCH - 1
    rd(last, 0).wait()
    pltpu.sync_copy(rr[0], agg_sh.at[idx_v.at[last]], add=True)
    plsc.subcore_barrier()


    pltpu.sync_copy(agg_sh.at[pl.ds(row0, ROWS_PER_TILE)],
                    out_hbm.at[c, pl.ds(row0, ROWS_PER_TILE)])


E_BLK = 8000


def _unpack_bf16(p):
    even = lax.bitcast_convert_type(p << 16, jnp.float32).astype(jnp.bfloat16)
    odd = lax.bitcast_convert_type(p & jnp.int32(-65536),
                                   jnp.float32).astype(jnp.bfloat16)
    return even, odd


def _edge_body(hsd_ref, ea_ref, eh_ref, w1s_ref, ct_ref, w2t_ref,
               b1_ref, b2_ref, g_ref, bb_ref, out_ref):
    hsd = hsd_ref[...]
    hse, hso = _unpack_bf16(hsd[:, :HP])
    hde, hdo = _unpack_bf16(hsd[:, HP:])
    hcat = jnp.concatenate([hse, hso, hde, hdo], axis=1)
    x = (jnp.dot(hcat, w1s_ref[...], preferred_element_type=jnp.float32)
         + jnp.dot(ea_ref[...], ct_ref[...], preferred_element_type=jnp.float32)
         + b1_ref[...])
    h = jnp.maximum(x, 0.0)
    h_bf = h.astype(jnp.bfloat16)
    msg = jnp.dot(h_bf, w2t_ref[...],
                  preferred_element_type=jnp.float32) + b2_ref[...]
    y = eh_ref[...] + msg
    mu = jnp.mean(y, axis=-1, keepdims=True)
    var = jnp.mean((y - mu) ** 2, axis=-1, keepdims=True)
    out_ref[...] = (y - mu) / jnp.sqrt(var + 1e-5) * g_ref[...] + bb_ref[...]


def _edge_mlp(hsd, ea, eh, w1s, ct, w2t, b1, b2, g, bb):
    grid = (N_EDGES // E_BLK,)
    full = lambda shape: pl.BlockSpec(shape, lambda i: (0, 0))
    return pl.pallas_call(
        _edge_body,
        grid=grid,
        in_specs=[
            pl.BlockSpec((E_BLK, H), lambda i: (i, 0)),
            pl.BlockSpec((E_BLK, EA), lambda i: (i, 0)),
            pl.BlockSpec((E_BLK, H), lambda i: (i, 0)),
            full((2 * H, 2 * H)),
            full((EA, 2 * H)),
            full((2 * H, H)),
            full((1, 2 * H)),
            full((1, H)),
            full((1, H)),
            full((1, H)),
        ],
        out_specs=pl.BlockSpec((E_BLK, H), lambda i: (i, 0)),
        out_shape=jax.ShapeDtypeStruct((N_EDGES, H), jnp.float32),
        compiler_params=pltpu.CompilerParams(
            dimension_semantics=("arbitrary",)),
    )(hsd, ea, eh, w1s, ct, w2t, b1, b2, g, bb)


N_BLK = 1000


def _node_body(nh_ref, a0_ref, a1_ref, dt_ref, et_ref, w2t_ref, b1_ref,
               b2_ref, g_ref, bb_ref, out_ref):
    agg = (a0_ref[...] + a1_ref[...]).astype(jnp.bfloat16)
    nh_bf = nh_ref[...].astype(jnp.bfloat16)
    x = (jnp.dot(nh_bf, dt_ref[...], preferred_element_type=jnp.float32)
         + jnp.dot(agg, et_ref[...], preferred_element_type=jnp.float32)
         + b1_ref[...])
    h = jnp.maximum(x, 0.0)
    upd = jnp.dot(h, w2t_ref[...], preferred_element_type=jnp.float32) + b2_ref[...]
    y = nh_ref[...] + upd
    mu = jnp.mean(y, axis=-1, keepdims=True)
    var = jnp.mean((y - mu) ** 2, axis=-1, keepdims=True)
    out_ref[...] = (y - mu) / jnp.sqrt(var + 1e-5) * g_ref[...] + bb_ref[...]


def _node_mlp(nh, a0, a1, dt, et, w2t, b1, b2, g, bb):
    grid = (N_NODES // N_BLK,)
    full = lambda shape: pl.BlockSpec(shape, lambda i: (0, 0))
    return pl.pallas_call(
        _node_body,
        grid=grid,
        in_specs=[
            pl.BlockSpec((N_BLK, H), lambda i: (i, 0)),
            pl.BlockSpec((N_BLK, H), lambda i: (i, 0)),
            pl.BlockSpec((N_BLK, H), lambda i: (i, 0)),
            full((H, 2 * H)),
            full((H, 2 * H)),
            full((2 * H, H)),
            full((1, 2 * H)),
            full((1, H)),
            full((1, H)),
            full((1, H)),
        ],
        out_specs=pl.BlockSpec((N_BLK, H), lambda i: (i, 0)),
        out_shape=jax.ShapeDtypeStruct((N_NODES, H), jnp.float32),
        compiler_params=pltpu.CompilerParams(
            dimension_semantics=("arbitrary",)),
    )(nh, a0, a1, dt, et, w2t, b1, b2, g, bb)


def kernel(node_h, edge_h, edge_index, edge_attr,
           W_e1, b_e1, W_e2, b_e2, W_n1, b_n1, W_n2, b_n2,
           ln_e_g, ln_e_b, ln_n_g, ln_n_b):
    ei = edge_index.astype(jnp.int32)
    src3 = ei[0].reshape(NW, NCH, CH)
    dst3 = ei[1].reshape(NW, NCH, CH)

    node_pack = lax.bitcast_convert_type(
        node_h.astype(jnp.bfloat16).reshape(N_NODES, HP, 2), jnp.int32)
    hsd = _sc_gather(node_pack, src3, dst3)

    at = W_e1[:, :H].T
    bt = W_e1[:, H:2 * H].T
    w1s = jnp.concatenate(
        [at[0::2], at[1::2], bt[0::2], bt[1::2]], axis=0).astype(jnp.bfloat16)
    ct = W_e1[:, 2 * H:].T.astype(jnp.bfloat16)
    w2t = W_e2.T.astype(jnp.bfloat16)
    ea_bf = edge_attr.astype(jnp.bfloat16)
    edge_h_new = _edge_mlp(hsd, ea_bf, edge_h, w1s, ct, w2t,
                           b_e1.reshape(1, -1), b_e2.reshape(1, -1),
                           ln_e_g.reshape(1, -1), ln_e_b.reshape(1, -1))

    zeros_pad = jnp.zeros((N_PAD, H), jnp.float32)
    parts = _sc_scatter(edge_h_new, dst3, zeros_pad)
    p0 = parts[0, :N_NODES]
    p1 = parts[1, :N_NODES]

    dt = W_n1[:, :H].T.astype(jnp.bfloat16)
    et = W_n1[:, H:].T.astype(jnp.bfloat16)
    wn2t = W_n2.T.astype(jnp.bfloat16)
    node_h_new = _node_mlp(node_h, p0, p1, dt, et, wn2t,
                           b_n1.reshape(1, -1), b_n2.reshape(1, -1),
                           ln_n_g.reshape(1, -1), ln_n_b.reshape(1, -1))
    return (node_h_new, edge_h_new)


# speedup vs baseline: 2.3428x; 1.0076x over previous
import functools

import jax
import jax.numpy as jnp
from jax import lax
from jax.experimental import pallas as pl
from jax.experimental.pallas import tpu as pltpu
from jax.experimental.pallas import tpu_sc as plsc

N_NODES = 10000
N_EDGES = 320000
H = 128
EA = 16

_INFO = plsc.get_sparse_core_info()
NC = _INFO.num_cores
NS = _INFO.num_subcores
NW = NC * NS
EPW = N_EDGES // NW
CH = 80
NCH = EPW // CH
N_PAD = 10240
ROWS_PER_TILE = N_PAD // NS

_mesh = plsc.VectorSubcoreMesh(core_axis_name="c", subcore_axis_name="s")


HP = H // 2


@functools.partial(
    pl.kernel,
    out_type=jax.ShapeDtypeStruct((N_EDGES, H), jnp.int32),
    mesh=_mesh,
    scratch_types=[
        pltpu.VMEM((NCH, CH), jnp.int32),
        pltpu.VMEM((NCH, CH), jnp.int32),
        pltpu.VMEM((CH, HP), jnp.int32),
        pltpu.VMEM((CH, HP), jnp.int32),
        pltpu.VMEM((CH, HP), jnp.int32),
        pltpu.VMEM((CH, HP), jnp.int32),
        pltpu.SemaphoreType.DMA,
        pltpu.SemaphoreType.DMA,
        pltpu.SemaphoreType.DMA,
        pltpu.SemaphoreType.DMA,
        pltpu.SemaphoreType.DMA,
        pltpu.SemaphoreType.DMA,
        pltpu.SemaphoreType.DMA,
        pltpu.SemaphoreType.DMA,
    ],
    compiler_params=pltpu.CompilerParams(use_tc_tiling_on_sc=False),
)
def _sc_gather(node_hbm, src3_hbm, dst3_hbm, hsd_hbm,
               idxs_v, idxd_v, rs0, rs1, rd0, rd1,
               gs0, gs1, gd0, gd1, ws0, ws1, wd0, wd1):
    c = lax.axis_index("c")
    s = lax.axis_index("s")
    wid = s * NC + c
    base_ch = wid * NCH
    pltpu.sync_copy(src3_hbm.at[wid], idxs_v)
    pltpu.sync_copy(dst3_hbm.at[wid], idxd_v)

    rs = (rs0, rs1)
    rd = (rd0, rd1)
    gs = (gs0, gs1)
    gd = (gd0, gd1)
    ws = (ws0, ws1)
    wd = (wd0, wd1)

    def wb_s(k, b):
        return pltpu.make_async_copy(
            rs[b], hsd_hbm.at[pl.ds((base_ch + k) * CH, CH), pl.ds(0, HP)],
            ws[b])

    def wb_d(k, b):
        return pltpu.make_async_copy(
            rd[b], hsd_hbm.at[pl.ds((base_ch + k) * CH, CH), pl.ds(HP, HP)],
            wd[b])

    pltpu.async_copy(node_hbm.at[idxs_v.at[0]], rs[0], gs[0])
    pltpu.async_copy(node_hbm.at[idxd_v.at[0]], rd[0], gd[0])

    @pl.loop(0, NCH - 1, step=2)
    def _pipe(j):
        for b in range(2):
            k = j + b
            pltpu.make_async_copy(node_hbm.at[idxs_v.at[k]], rs[b], gs[b]).wait()
            pltpu.make_async_copy(node_hbm.at[idxd_v.at[k]], rd[b], gd[b]).wait()
            if b == 1:
                wb_s(k - 1, 0).wait()
                wb_d(k - 1, 0).wait()
            else:
                @pl.when(j >= 1)
                def _():
                    wb_s(k - 1, 1).wait()
                    wb_d(k - 1, 1).wait()
            pltpu.async_copy(node_hbm.at[idxs_v.at[k + 1]], rs[1 - b], gs[1 - b])
            pltpu.async_copy(node_hbm.at[idxd_v.at[k + 1]], rd[1 - b], gd[1 - b])
            wb_s(k, b).start()
            wb_d(k, b).start()

    last = NCH - 1
    pltpu.make_async_copy(node_hbm.at[idxs_v.at[last]], rs[0], gs[0]).wait()
    pltpu.make_async_copy(node_hbm.at[idxd_v.at[last]], rd[0], gd[0]).wait()
    wb_s(last - 1, 1).wait()
    wb_d(last - 1, 1).wait()
    wb_s(last, 0).start()
    wb_d(last, 0).start()
    wb_s(last, 0).wait()
    wb_d(last, 0).wait()


@functools.partial(
    pl.kernel,
    out_type=jax.ShapeDtypeStruct((NC, N_PAD, H), jnp.float32),
    mesh=_mesh,
    scratch_types=[
        pltpu.VMEM((NCH, CH), jnp.int32),
        pltpu.VMEM((CH, H), jnp.float32),
        pltpu.VMEM((CH, H), jnp.float32),
        pltpu.VMEM_SHARED((N_PAD, H), jnp.float32),
        pltpu.SemaphoreType.DMA,
        pltpu.SemaphoreType.DMA,
    ],
)
def _sc_scatter(ehn_hbm, dst3_hbm, zeros_hbm, out_hbm,
                idx_v, r0, r1, agg_sh, rs0, rs1):
    c = lax.axis_index("c")
    s = lax.axis_index("s")
    wid = s * NC + c
    base_ch = wid * NCH

    row0 = s * ROWS_PER_TILE
    pltpu.sync_copy(zeros_hbm.at[pl.ds(row0, ROWS_PER_TILE)],
                    agg_sh.at[pl.ds(row0, ROWS_PER_TILE)])
    plsc.subcore_barrier()

    pltpu.sync_copy(dst3_hbm.at[wid], idx_v)

    rr = (r0, r1)
    ss = (rs0, rs1)

    def rd(k, b):
        return pltpu.make_async_copy(
            ehn_hbm.at[pl.ds((base_ch + k) * CH, CH)], rr[b], ss[b])

    rd(0, 0).start()

    @pl.loop(0, NCH - 1, step=2)
    def _pipe(j):
        for b in range(2):
            k = j + b
            rd(k, b).wait()
            rd(k + 1, 1 - b).start()
            pltpu.sync_copy(rr[b], agg_sh.at[idx_v.at[k]], add=True)

    last = NCH - 1
    rd(last, 0).wait()
    pltpu.sync_copy(rr[0], agg_sh.at[idx_v.at[last]], add=True)
    plsc.subcore_barrier()


    pltpu.sync_copy(agg_sh.at[pl.ds(row0, ROWS_PER_TILE)],
                    out_hbm.at[c, pl.ds(row0, ROWS_PER_TILE)])


E_BLK = 10000


def _unpack_bf16(p):
    even = lax.bitcast_convert_type(p << 16, jnp.float32).astype(jnp.bfloat16)
    odd = lax.bitcast_convert_type(p & jnp.int32(-65536),
                                   jnp.float32).astype(jnp.bfloat16)
    return even, odd


def _edge_body(hsd_ref, ea_ref, eh_ref, w1s_ref, ct_ref, w2t_ref,
               b1_ref, b2_ref, g_ref, bb_ref, out_ref):
    hsd = hsd_ref[...]
    hse, hso = _unpack_bf16(hsd[:, :HP])
    hde, hdo = _unpack_bf16(hsd[:, HP:])
    hcat = jnp.concatenate([hse, hso, hde, hdo], axis=1)
    x = (jnp.dot(hcat, w1s_ref[...], preferred_element_type=jnp.float32)
         + jnp.dot(ea_ref[...], ct_ref[...], preferred_element_type=jnp.float32)
         + b1_ref[...])
    h = jnp.maximum(x, 0.0)
    h_bf = h.astype(jnp.bfloat16)
    msg = jnp.dot(h_bf, w2t_ref[...],
                  preferred_element_type=jnp.float32) + b2_ref[...]
    y = eh_ref[...] + msg
    mu = jnp.mean(y, axis=-1, keepdims=True)
    var = jnp.mean((y - mu) ** 2, axis=-1, keepdims=True)
    out_ref[...] = (y - mu) / jnp.sqrt(var + 1e-5) * g_ref[...] + bb_ref[...]


def _edge_mlp(hsd, ea, eh, w1s, ct, w2t, b1, b2, g, bb):
    grid = (N_EDGES // E_BLK,)
    full = lambda shape: pl.BlockSpec(shape, lambda i: (0, 0))
    return pl.pallas_call(
        _edge_body,
        grid=grid,
        in_specs=[
            pl.BlockSpec((E_BLK, H), lambda i: (i, 0)),
            pl.BlockSpec((E_BLK, EA), lambda i: (i, 0)),
            pl.BlockSpec((E_BLK, H), lambda i: (i, 0)),
            full((2 * H, 2 * H)),
            full((EA, 2 * H)),
            full((2 * H, H)),
            full((1, 2 * H)),
            full((1, H)),
            full((1, H)),
            full((1, H)),
        ],
        out_specs=pl.BlockSpec((E_BLK, H), lambda i: (i, 0)),
        out_shape=jax.ShapeDtypeStruct((N_EDGES, H), jnp.float32),
        compiler_params=pltpu.CompilerParams(
            dimension_semantics=("arbitrary",)),
    )(hsd, ea, eh, w1s, ct, w2t, b1, b2, g, bb)


N_BLK = 1000


def _node_body(nh_ref, a0_ref, a1_ref, dt_ref, et_ref, w2t_ref, b1_ref,
               b2_ref, g_ref, bb_ref, out_ref):
    agg = (a0_ref[...] + a1_ref[...]).astype(jnp.bfloat16)
    nh_bf = nh_ref[...].astype(jnp.bfloat16)
    x = (jnp.dot(nh_bf, dt_ref[...], preferred_element_type=jnp.float32)
         + jnp.dot(agg, et_ref[...], preferred_element_type=jnp.float32)
         + b1_ref[...])
    h = jnp.maximum(x, 0.0)
    upd = jnp.dot(h, w2t_ref[...], preferred_element_type=jnp.float32) + b2_ref[...]
    y = nh_ref[...] + upd
    mu = jnp.mean(y, axis=-1, keepdims=True)
    var = jnp.mean((y - mu) ** 2, axis=-1, keepdims=True)
    out_ref[...] = (y - mu) / jnp.sqrt(var + 1e-5) * g_ref[...] + bb_ref[...]


def _node_mlp(nh, a0, a1, dt, et, w2t, b1, b2, g, bb):
    grid = (N_NODES // N_BLK,)
    full = lambda shape: pl.BlockSpec(shape, lambda i: (0, 0))
    return pl.pallas_call(
        _node_body,
        grid=grid,
        in_specs=[
            pl.BlockSpec((N_BLK, H), lambda i: (i, 0)),
            pl.BlockSpec((N_BLK, H), lambda i: (i, 0)),
            pl.BlockSpec((N_BLK, H), lambda i: (i, 0)),
            full((H, 2 * H)),
            full((H, 2 * H)),
            full((2 * H, H)),
            full((1, 2 * H)),
            full((1, H)),
            full((1, H)),
            full((1, H)),
        ],
        out_specs=pl.BlockSpec((N_BLK, H), lambda i: (i, 0)),
        out_shape=jax.ShapeDtypeStruct((N_NODES, H), jnp.float32),
        compiler_params=pltpu.CompilerParams(
            dimension_semantics=("arbitrary",)),
    )(nh, a0, a1, dt, et, w2t, b1, b2, g, bb)


def kernel(node_h, edge_h, edge_index, edge_attr,
           W_e1, b_e1, W_e2, b_e2, W_n1, b_n1, W_n2, b_n2,
           ln_e_g, ln_e_b, ln_n_g, ln_n_b):
    ei = edge_index.astype(jnp.int32)
    src3 = ei[0].reshape(NW, NCH, CH)
    dst3 = ei[1].reshape(NW, NCH, CH)

    node_pack = lax.bitcast_convert_type(
        node_h.astype(jnp.bfloat16).reshape(N_NODES, HP, 2), jnp.int32)
    hsd = _sc_gather(node_pack, src3, dst3)

    at = W_e1[:, :H].T
    bt = W_e1[:, H:2 * H].T
    w1s = jnp.concatenate(
        [at[0::2], at[1::2], bt[0::2], bt[1::2]], axis=0).astype(jnp.bfloat16)
    ct = W_e1[:, 2 * H:].T.astype(jnp.bfloat16)
    w2t = W_e2.T.astype(jnp.bfloat16)
    ea_bf = edge_attr.astype(jnp.bfloat16)
    edge_h_new = _edge_mlp(hsd, ea_bf, edge_h, w1s, ct, w2t,
                           b_e1.reshape(1, -1), b_e2.reshape(1, -1),
                           ln_e_g.reshape(1, -1), ln_e_b.reshape(1, -1))

    zeros_pad = jnp.zeros((N_PAD, H), jnp.float32)
    parts = _sc_scatter(edge_h_new, dst3, zeros_pad)
    p0 = parts[0, :N_NODES]
    p1 = parts[1, :N_NODES]

    dt = W_n1[:, :H].T.astype(jnp.bfloat16)
    et = W_n1[:, H:].T.astype(jnp.bfloat16)
    wn2t = W_n2.T.astype(jnp.bfloat16)
    node_h_new = _node_mlp(node_h, p0, p1, dt, et, wn2t,
                           b_n1.reshape(1, -1), b_n2.reshape(1, -1),
                           ln_n_g.reshape(1, -1), ln_n_b.reshape(1, -1))
    return (node_h_new, edge_h_new)


# speedup vs baseline: 2.6686x; 1.1391x over previous
import functools

import jax
import jax.numpy as jnp
from jax import lax
from jax.experimental import pallas as pl
from jax.experimental.pallas import tpu as pltpu
from jax.experimental.pallas import tpu_sc as plsc

N_NODES = 10000
N_EDGES = 320000
H = 128
EA = 16

_INFO = plsc.get_sparse_core_info()
NC = _INFO.num_cores
NS = _INFO.num_subcores
NW = NC * NS
EPW = N_EDGES // NW
CH = 80
NCH = EPW // CH
N_PAD = 10240
ROWS_PER_TILE = N_PAD // NS

_mesh = plsc.VectorSubcoreMesh(core_axis_name="c", subcore_axis_name="s")


HP = H // 2


@functools.partial(
    pl.kernel,
    out_type=jax.ShapeDtypeStruct((N_EDGES, H), jnp.int32),
    mesh=_mesh,
    scratch_types=[
        pltpu.VMEM((NCH, CH), jnp.int32),
        pltpu.VMEM((NCH, CH), jnp.int32),
        pltpu.VMEM((CH, HP), jnp.int32),
        pltpu.VMEM((CH, HP), jnp.int32),
        pltpu.VMEM((CH, HP), jnp.int32),
        pltpu.VMEM((CH, HP), jnp.int32),
        pltpu.VMEM_SHARED((N_PAD, HP), jnp.int32),
        pltpu.SemaphoreType.DMA,
        pltpu.SemaphoreType.DMA,
        pltpu.SemaphoreType.DMA,
        pltpu.SemaphoreType.DMA,
        pltpu.SemaphoreType.DMA,
        pltpu.SemaphoreType.DMA,
        pltpu.SemaphoreType.DMA,
        pltpu.SemaphoreType.DMA,
    ],
    compiler_params=pltpu.CompilerParams(use_tc_tiling_on_sc=False),
)
def _sc_gather(node_hbm, src3_hbm, dst3_hbm, hsd_hbm,
               idxs_v, idxd_v, rs0, rs1, rd0, rd1, tab_sh,
               gs0, gs1, gd0, gd1, ws0, ws1, wd0, wd1):
    c = lax.axis_index("c")
    s = lax.axis_index("s")
    wid = s * NC + c
    base_ch = wid * NCH
    trow = s * (N_PAD // NS)
    pltpu.sync_copy(node_hbm.at[pl.ds(trow, N_PAD // NS)],
                    tab_sh.at[pl.ds(trow, N_PAD // NS)])
    pltpu.sync_copy(src3_hbm.at[wid], idxs_v)
    pltpu.sync_copy(dst3_hbm.at[wid], idxd_v)
    plsc.subcore_barrier()

    rs = (rs0, rs1)
    rd = (rd0, rd1)
    gs = (gs0, gs1)
    gd = (gd0, gd1)
    ws = (ws0, ws1)
    wd = (wd0, wd1)

    def wb_s(k, b):
        return pltpu.make_async_copy(
            rs[b], hsd_hbm.at[pl.ds((base_ch + k) * CH, CH), pl.ds(0, HP)],
            ws[b])

    def wb_d(k, b):
        return pltpu.make_async_copy(
            rd[b], hsd_hbm.at[pl.ds((base_ch + k) * CH, CH), pl.ds(HP, HP)],
            wd[b])

    pltpu.async_copy(tab_sh.at[idxs_v.at[0]], rs[0], gs[0])
    pltpu.async_copy(tab_sh.at[idxd_v.at[0]], rd[0], gd[0])

    @pl.loop(0, NCH - 1, step=2)
    def _pipe(j):
        for b in range(2):
            k = j + b
            pltpu.make_async_copy(tab_sh.at[idxs_v.at[k]], rs[b], gs[b]).wait()
            pltpu.make_async_copy(tab_sh.at[idxd_v.at[k]], rd[b], gd[b]).wait()
            if b == 1:
                wb_s(k - 1, 0).wait()
                wb_d(k - 1, 0).wait()
            else:
                @pl.when(j >= 1)
                def _():
                    wb_s(k - 1, 1).wait()
                    wb_d(k - 1, 1).wait()
            pltpu.async_copy(tab_sh.at[idxs_v.at[k + 1]], rs[1 - b], gs[1 - b])
            pltpu.async_copy(tab_sh.at[idxd_v.at[k + 1]], rd[1 - b], gd[1 - b])
            wb_s(k, b).start()
            wb_d(k, b).start()

    last = NCH - 1
    pltpu.make_async_copy(tab_sh.at[idxs_v.at[last]], rs[0], gs[0]).wait()
    pltpu.make_async_copy(tab_sh.at[idxd_v.at[last]], rd[0], gd[0]).wait()
    wb_s(last - 1, 1).wait()
    wb_d(last - 1, 1).wait()
    wb_s(last, 0).start()
    wb_d(last, 0).start()
    wb_s(last, 0).wait()
    wb_d(last, 0).wait()


@functools.partial(
    pl.kernel,
    out_type=jax.ShapeDtypeStruct((NC, N_PAD, H), jnp.float32),
    mesh=_mesh,
    scratch_types=[
        pltpu.VMEM((NCH, CH), jnp.int32),
        pltpu.VMEM((CH, H), jnp.float32),
        pltpu.VMEM((CH, H), jnp.float32),
        pltpu.VMEM_SHARED((N_PAD, H), jnp.float32),
        pltpu.SemaphoreType.DMA,
        pltpu.SemaphoreType.DMA,
    ],
)
def _sc_scatter(ehn_hbm, dst3_hbm, zeros_hbm, out_hbm,
                idx_v, r0, r1, agg_sh, rs0, rs1):
    c = lax.axis_index("c")
    s = lax.axis_index("s")
    wid = s * NC + c
    base_ch = wid * NCH

    row0 = s * ROWS_PER_TILE
    pltpu.sync_copy(zeros_hbm.at[pl.ds(row0, ROWS_PER_TILE)],
                    agg_sh.at[pl.ds(row0, ROWS_PER_TILE)])
    plsc.subcore_barrier()

    pltpu.sync_copy(dst3_hbm.at[wid], idx_v)

    rr = (r0, r1)
    ss = (rs0, rs1)

    def rd(k, b):
        return pltpu.make_async_copy(
            ehn_hbm.at[pl.ds((base_ch + k) * CH, CH)], rr[b], ss[b])

    rd(0, 0).start()

    @pl.loop(0, NCH - 1, step=2)
    def _pipe(j):
        for b in range(2):
            k = j + b
            rd(k, b).wait()
            rd(k + 1, 1 - b).start()
            pltpu.sync_copy(rr[b], agg_sh.at[idx_v.at[k]], add=True)

    last = NCH - 1
    rd(last, 0).wait()
    pltpu.sync_copy(rr[0], agg_sh.at[idx_v.at[last]], add=True)
    plsc.subcore_barrier()


    pltpu.sync_copy(agg_sh.at[pl.ds(row0, ROWS_PER_TILE)],
                    out_hbm.at[c, pl.ds(row0, ROWS_PER_TILE)])


E_BLK = 10000


def _unpack_bf16(p):
    even = lax.bitcast_convert_type(p << 16, jnp.float32).astype(jnp.bfloat16)
    odd = lax.bitcast_convert_type(p & jnp.int32(-65536),
                                   jnp.float32).astype(jnp.bfloat16)
    return even, odd


def _edge_body(hsd_ref, ea_ref, eh_ref, w1s_ref, ct_ref, w2t_ref,
               b1_ref, b2_ref, g_ref, bb_ref, out_ref):
    hsd = hsd_ref[...]
    hse, hso = _unpack_bf16(hsd[:, :HP])
    hde, hdo = _unpack_bf16(hsd[:, HP:])
    hcat = jnp.concatenate([hse, hso, hde, hdo], axis=1)
    x = (jnp.dot(hcat, w1s_ref[...], preferred_element_type=jnp.float32)
         + jnp.dot(ea_ref[...], ct_ref[...], preferred_element_type=jnp.float32)
         + b1_ref[...])
    h = jnp.maximum(x, 0.0)
    h_bf = h.astype(jnp.bfloat16)
    msg = jnp.dot(h_bf, w2t_ref[...],
                  preferred_element_type=jnp.float32) + b2_ref[...]
    y = eh_ref[...] + msg
    mu = jnp.mean(y, axis=-1, keepdims=True)
    var = jnp.mean((y - mu) ** 2, axis=-1, keepdims=True)
    out_ref[...] = (y - mu) / jnp.sqrt(var + 1e-5) * g_ref[...] + bb_ref[...]


def _edge_mlp(hsd, ea, eh, w1s, ct, w2t, b1, b2, g, bb):
    grid = (N_EDGES // E_BLK,)
    full = lambda shape: pl.BlockSpec(shape, lambda i: (0, 0))
    return pl.pallas_call(
        _edge_body,
        grid=grid,
        in_specs=[
            pl.BlockSpec((E_BLK, H), lambda i: (i, 0)),
            pl.BlockSpec((E_BLK, EA), lambda i: (i, 0)),
            pl.BlockSpec((E_BLK, H), lambda i: (i, 0)),
            full((2 * H, 2 * H)),
            full((EA, 2 * H)),
            full((2 * H, H)),
            full((1, 2 * H)),
            full((1, H)),
            full((1, H)),
            full((1, H)),
        ],
        out_specs=pl.BlockSpec((E_BLK, H), lambda i: (i, 0)),
        out_shape=jax.ShapeDtypeStruct((N_EDGES, H), jnp.float32),
        compiler_params=pltpu.CompilerParams(
            dimension_semantics=("arbitrary",)),
    )(hsd, ea, eh, w1s, ct, w2t, b1, b2, g, bb)


N_BLK = 1000


def _node_body(nh_ref, a0_ref, a1_ref, dt_ref, et_ref, w2t_ref, b1_ref,
               b2_ref, g_ref, bb_ref, out_ref):
    agg = (a0_ref[...] + a1_ref[...]).astype(jnp.bfloat16)
    nh_bf = nh_ref[...].astype(jnp.bfloat16)
    x = (jnp.dot(nh_bf, dt_ref[...], preferred_element_type=jnp.float32)
         + jnp.dot(agg, et_ref[...], preferred_element_type=jnp.float32)
         + b1_ref[...])
    h = jnp.maximum(x, 0.0)
    upd = jnp.dot(h, w2t_ref[...], preferred_element_type=jnp.float32) + b2_ref[...]
    y = nh_ref[...] + upd
    mu = jnp.mean(y, axis=-1, keepdims=True)
    var = jnp.mean((y - mu) ** 2, axis=-1, keepdims=True)
    out_ref[...] = (y - mu) / jnp.sqrt(var + 1e-5) * g_ref[...] + bb_ref[...]


def _node_mlp(nh, a0, a1, dt, et, w2t, b1, b2, g, bb):
    grid = (N_NODES // N_BLK,)
    full = lambda shape: pl.BlockSpec(shape, lambda i: (0, 0))
    return pl.pallas_call(
        _node_body,
        grid=grid,
        in_specs=[
            pl.BlockSpec((N_BLK, H), lambda i: (i, 0)),
            pl.BlockSpec((N_BLK, H), lambda i: (i, 0)),
            pl.BlockSpec((N_BLK, H), lambda i: (i, 0)),
            full((H, 2 * H)),
            full((H, 2 * H)),
            full((2 * H, H)),
            full((1, 2 * H)),
            full((1, H)),
            full((1, H)),
            full((1, H)),
        ],
        out_specs=pl.BlockSpec((N_BLK, H), lambda i: (i, 0)),
        out_shape=jax.ShapeDtypeStruct((N_NODES, H), jnp.float32),
        compiler_params=pltpu.CompilerParams(
            dimension_semantics=("arbitrary",)),
    )(nh, a0, a1, dt, et, w2t, b1, b2, g, bb)


def kernel(node_h, edge_h, edge_index, edge_attr,
           W_e1, b_e1, W_e2, b_e2, W_n1, b_n1, W_n2, b_n2,
           ln_e_g, ln_e_b, ln_n_g, ln_n_b):
    ei = edge_index.astype(jnp.int32)
    src3 = ei[0].reshape(NW, NCH, CH)
    dst3 = ei[1].reshape(NW, NCH, CH)

    node_pack = lax.bitcast_convert_type(
        node_h.astype(jnp.bfloat16).reshape(N_NODES, HP, 2), jnp.int32)
    node_pack = jnp.pad(node_pack, ((0, N_PAD - N_NODES), (0, 0)))
    hsd = _sc_gather(node_pack, src3, dst3)

    at = W_e1[:, :H].T
    bt = W_e1[:, H:2 * H].T
    w1s = jnp.concatenate(
        [at[0::2], at[1::2], bt[0::2], bt[1::2]], axis=0).astype(jnp.bfloat16)
    ct = W_e1[:, 2 * H:].T.astype(jnp.bfloat16)
    w2t = W_e2.T.astype(jnp.bfloat16)
    ea_bf = edge_attr.astype(jnp.bfloat16)
    edge_h_new = _edge_mlp(hsd, ea_bf, edge_h, w1s, ct, w2t,
                           b_e1.reshape(1, -1), b_e2.reshape(1, -1),
                           ln_e_g.reshape(1, -1), ln_e_b.reshape(1, -1))

    zeros_pad = jnp.zeros((N_PAD, H), jnp.float32)
    parts = _sc_scatter(edge_h_new, dst3, zeros_pad)
    p0 = parts[0, :N_NODES]
    p1 = parts[1, :N_NODES]

    dt = W_n1[:, :H].T.astype(jnp.bfloat16)
    et = W_n1[:, H:].T.astype(jnp.bfloat16)
    wn2t = W_n2.T.astype(jnp.bfloat16)
    node_h_new = _node_mlp(node_h, p0, p1, dt, et, wn2t,
                           b_n1.reshape(1, -1), b_n2.reshape(1, -1),
                           ln_n_g.reshape(1, -1), ln_n_b.reshape(1, -1))
    return (node_h_new, edge_h_new)


# speedup vs baseline: 3.0045x; 1.1259x over previous
import functools

import jax
import jax.numpy as jnp
from jax import lax
from jax.experimental import pallas as pl
from jax.experimental.pallas import tpu as pltpu
from jax.experimental.pallas import tpu_sc as plsc

N_NODES = 10000
N_EDGES = 320000
H = 128
EA = 16

_INFO = plsc.get_sparse_core_info()
NC = _INFO.num_cores
NS = _INFO.num_subcores
NW = NC * NS
EPW = N_EDGES // NW
CH = 80
NCH = EPW // CH
ROWS_PER_TILE = N_NODES // NS

_mesh = plsc.VectorSubcoreMesh(core_axis_name="c", subcore_axis_name="s")


HP = H // 2


@functools.partial(
    pl.kernel,
    out_type=jax.ShapeDtypeStruct((N_EDGES, H), jnp.int32),
    mesh=_mesh,
    scratch_types=[
        pltpu.VMEM((NCH, CH), jnp.int32),
        pltpu.VMEM((NCH, CH), jnp.int32),
        pltpu.VMEM((CH, HP), jnp.int32),
        pltpu.VMEM((CH, HP), jnp.int32),
        pltpu.VMEM((CH, HP), jnp.int32),
        pltpu.VMEM((CH, HP), jnp.int32),
        pltpu.VMEM_SHARED((N_NODES, HP), jnp.int32),
        pltpu.SemaphoreType.DMA,
        pltpu.SemaphoreType.DMA,
        pltpu.SemaphoreType.DMA,
        pltpu.SemaphoreType.DMA,
        pltpu.SemaphoreType.DMA,
        pltpu.SemaphoreType.DMA,
        pltpu.SemaphoreType.DMA,
        pltpu.SemaphoreType.DMA,
    ],
    compiler_params=pltpu.CompilerParams(use_tc_tiling_on_sc=False),
)
def _sc_gather(node_hbm, src3_hbm, dst3_hbm, hsd_hbm,
               idxs_v, idxd_v, rs0, rs1, rd0, rd1, tab_sh,
               gs0, gs1, gd0, gd1, ws0, ws1, wd0, wd1):
    c = lax.axis_index("c")
    s = lax.axis_index("s")
    wid = s * NC + c
    base_ch = wid * NCH
    trow = s * (N_NODES // NS)
    pltpu.sync_copy(node_hbm.at[pl.ds(trow, N_NODES // NS)],
                    tab_sh.at[pl.ds(trow, N_NODES // NS)])
    pltpu.sync_copy(src3_hbm.at[wid], idxs_v)
    pltpu.sync_copy(dst3_hbm.at[wid], idxd_v)
    plsc.subcore_barrier()

    rs = (rs0, rs1)
    rd = (rd0, rd1)
    gs = (gs0, gs1)
    gd = (gd0, gd1)
    ws = (ws0, ws1)
    wd = (wd0, wd1)

    def wb_s(k, b):
        return pltpu.make_async_copy(
            rs[b], hsd_hbm.at[pl.ds((base_ch + k) * CH, CH), pl.ds(0, HP)],
            ws[b])

    def wb_d(k, b):
        return pltpu.make_async_copy(
            rd[b], hsd_hbm.at[pl.ds((base_ch + k) * CH, CH), pl.ds(HP, HP)],
            wd[b])

    pltpu.async_copy(tab_sh.at[idxs_v.at[0]], rs[0], gs[0])
    pltpu.async_copy(tab_sh.at[idxd_v.at[0]], rd[0], gd[0])

    @pl.loop(0, NCH - 1, step=2)
    def _pipe(j):
        for b in range(2):
            k = j + b
            pltpu.make_async_copy(tab_sh.at[idxs_v.at[k]], rs[b], gs[b]).wait()
            pltpu.make_async_copy(tab_sh.at[idxd_v.at[k]], rd[b], gd[b]).wait()
            if b == 1:
                wb_s(k - 1, 0).wait()
                wb_d(k - 1, 0).wait()
            else:
                @pl.when(j >= 1)
                def _():
                    wb_s(k - 1, 1).wait()
                    wb_d(k - 1, 1).wait()
            pltpu.async_copy(tab_sh.at[idxs_v.at[k + 1]], rs[1 - b], gs[1 - b])
            pltpu.async_copy(tab_sh.at[idxd_v.at[k + 1]], rd[1 - b], gd[1 - b])
            wb_s(k, b).start()
            wb_d(k, b).start()

    last = NCH - 1
    pltpu.make_async_copy(tab_sh.at[idxs_v.at[last]], rs[0], gs[0]).wait()
    pltpu.make_async_copy(tab_sh.at[idxd_v.at[last]], rd[0], gd[0]).wait()
    wb_s(last - 1, 1).wait()
    wb_d(last - 1, 1).wait()
    wb_s(last, 0).start()
    wb_d(last, 0).start()
    wb_s(last, 0).wait()
    wb_d(last, 0).wait()


@functools.partial(
    pl.kernel,
    out_type=jax.ShapeDtypeStruct((NC, N_NODES, H), jnp.float32),
    mesh=_mesh,
    scratch_types=[
        pltpu.VMEM((NCH, CH), jnp.int32),
        pltpu.VMEM((CH, H), jnp.float32),
        pltpu.VMEM((CH, H), jnp.float32),
        pltpu.VMEM((CH, H), jnp.float32),
        pltpu.VMEM((CH, H), jnp.float32),
        pltpu.VMEM_SHARED((N_NODES, H), jnp.float32),
        pltpu.SemaphoreType.DMA,
        pltpu.SemaphoreType.DMA,
        pltpu.SemaphoreType.DMA,
        pltpu.SemaphoreType.DMA,
    ],
    compiler_params=pltpu.CompilerParams(use_tc_tiling_on_sc=False),
)
def _sc_scatter(ehn_hbm, dst3_hbm, zeros_hbm, out_hbm,
                idx_v, r0, r1, r2, r3, agg_sh, rs0, rs1, rs2, rs3):
    c = lax.axis_index("c")
    s = lax.axis_index("s")
    wid = s * NC + c
    base_ch = wid * NCH

    row0 = s * ROWS_PER_TILE
    pltpu.sync_copy(zeros_hbm.at[pl.ds(row0, ROWS_PER_TILE)],
                    agg_sh.at[pl.ds(row0, ROWS_PER_TILE)])
    plsc.subcore_barrier()

    pltpu.sync_copy(dst3_hbm.at[wid], idx_v)

    rr = (r0, r1, r2, r3)
    ss = (rs0, rs1, rs2, rs3)

    def rd(k, b):
        return pltpu.make_async_copy(
            ehn_hbm.at[pl.ds((base_ch + k) * CH, CH)], rr[b], ss[b])

    rd(0, 0).start()
    rd(1, 1).start()
    rd(2, 2).start()

    @pl.loop(0, NCH - 1, step=4)
    def _pipe(j):
        for b in range(4):
            k = j + b
            rd(k, b).wait()

            @pl.when(k + 3 < NCH)
            def _():
                rd(k + 3, (b + 3) % 4).start()

            pltpu.sync_copy(rr[b], agg_sh.at[idx_v.at[k]], add=True)

    last = NCH - 1
    rd(last, 0).wait()
    pltpu.sync_copy(rr[0], agg_sh.at[idx_v.at[last]], add=True)
    plsc.subcore_barrier()


    pltpu.sync_copy(agg_sh.at[pl.ds(row0, ROWS_PER_TILE)],
                    out_hbm.at[c, pl.ds(row0, ROWS_PER_TILE)])


E_BLK = 10000


def _unpack_bf16(p):
    even = lax.bitcast_convert_type(p << 16, jnp.float32).astype(jnp.bfloat16)
    odd = lax.bitcast_convert_type(p & jnp.int32(-65536),
                                   jnp.float32).astype(jnp.bfloat16)
    return even, odd


def _edge_body(hsd_ref, ea_ref, eh_ref, w1s_ref, ct_ref, w2t_ref,
               b1_ref, b2_ref, g_ref, bb_ref, out_ref):
    hsd = hsd_ref[...]
    hse, hso = _unpack_bf16(hsd[:, :HP])
    hde, hdo = _unpack_bf16(hsd[:, HP:])
    hcat = jnp.concatenate([hse, hso, hde, hdo], axis=1)
    x = (jnp.dot(hcat, w1s_ref[...], preferred_element_type=jnp.float32)
         + jnp.dot(ea_ref[...], ct_ref[...], preferred_element_type=jnp.float32)
         + b1_ref[...])
    h = jnp.maximum(x, 0.0)
    h_bf = h.astype(jnp.bfloat16)
    msg = jnp.dot(h_bf, w2t_ref[...],
                  preferred_element_type=jnp.float32) + b2_ref[...]
    y = eh_ref[...] + msg
    mu = jnp.mean(y, axis=-1, keepdims=True)
    var = jnp.mean((y - mu) ** 2, axis=-1, keepdims=True)
    out_ref[...] = (y - mu) / jnp.sqrt(var + 1e-5) * g_ref[...] + bb_ref[...]


def _edge_mlp(hsd, ea, eh, w1s, ct, w2t, b1, b2, g, bb):
    grid = (N_EDGES // E_BLK,)
    full = lambda shape: pl.BlockSpec(shape, lambda i: (0, 0))
    return pl.pallas_call(
        _edge_body,
        grid=grid,
        in_specs=[
            pl.BlockSpec((E_BLK, H), lambda i: (i, 0)),
            pl.BlockSpec((E_BLK, EA), lambda i: (i, 0)),
            pl.BlockSpec((E_BLK, H), lambda i: (i, 0)),
            full((2 * H, 2 * H)),
            full((EA, 2 * H)),
            full((2 * H, H)),
            full((1, 2 * H)),
            full((1, H)),
            full((1, H)),
            full((1, H)),
        ],
        out_specs=pl.BlockSpec((E_BLK, H), lambda i: (i, 0)),
        out_shape=jax.ShapeDtypeStruct((N_EDGES, H), jnp.float32),
        compiler_params=pltpu.CompilerParams(
            dimension_semantics=("arbitrary",)),
    )(hsd, ea, eh, w1s, ct, w2t, b1, b2, g, bb)


N_BLK = 1000


def _node_body(nh_ref, a0_ref, a1_ref, dt_ref, et_ref, w2t_ref, b1_ref,
               b2_ref, g_ref, bb_ref, out_ref):
    agg = (a0_ref[...] + a1_ref[...]).astype(jnp.bfloat16)
    nh_bf = nh_ref[...].astype(jnp.bfloat16)
    x = (jnp.dot(nh_bf, dt_ref[...], preferred_element_type=jnp.float32)
         + jnp.dot(agg, et_ref[...], preferred_element_type=jnp.float32)
         + b1_ref[...])
    h = jnp.maximum(x, 0.0)
    upd = jnp.dot(h, w2t_ref[...], preferred_element_type=jnp.float32) + b2_ref[...]
    y = nh_ref[...] + upd
    mu = jnp.mean(y, axis=-1, keepdims=True)
    var = jnp.mean((y - mu) ** 2, axis=-1, keepdims=True)
    out_ref[...] = (y - mu) / jnp.sqrt(var + 1e-5) * g_ref[...] + bb_ref[...]


def _node_mlp(nh, a0, a1, dt, et, w2t, b1, b2, g, bb):
    grid = (N_NODES // N_BLK,)
    full = lambda shape: pl.BlockSpec(shape, lambda i: (0, 0))
    return pl.pallas_call(
        _node_body,
        grid=grid,
        in_specs=[
            pl.BlockSpec((N_BLK, H), lambda i: (i, 0)),
            pl.BlockSpec((N_BLK, H), lambda i: (i, 0)),
            pl.BlockSpec((N_BLK, H), lambda i: (i, 0)),
            full((H, 2 * H)),
            full((H, 2 * H)),
            full((2 * H, H)),
            full((1, 2 * H)),
            full((1, H)),
            full((1, H)),
            full((1, H)),
        ],
        out_specs=pl.BlockSpec((N_BLK, H), lambda i: (i, 0)),
        out_shape=jax.ShapeDtypeStruct((N_NODES, H), jnp.float32),
        compiler_params=pltpu.CompilerParams(
            dimension_semantics=("arbitrary",)),
    )(nh, a0, a1, dt, et, w2t, b1, b2, g, bb)


def kernel(node_h, edge_h, edge_index, edge_attr,
           W_e1, b_e1, W_e2, b_e2, W_n1, b_n1, W_n2, b_n2,
           ln_e_g, ln_e_b, ln_n_g, ln_n_b):
    ei = edge_index.astype(jnp.int32)
    src3 = ei[0].reshape(NW, NCH, CH)
    dst3 = ei[1].reshape(NW, NCH, CH)

    node_pack = lax.bitcast_convert_type(
        node_h.astype(jnp.bfloat16).reshape(N_NODES, HP, 2), jnp.int32)
    hsd = _sc_gather(node_pack, src3, dst3)

    at = W_e1[:, :H].T
    bt = W_e1[:, H:2 * H].T
    w1s = jnp.concatenate(
        [at[0::2], at[1::2], bt[0::2], bt[1::2]], axis=0).astype(jnp.bfloat16)
    ct = W_e1[:, 2 * H:].T.astype(jnp.bfloat16)
    w2t = W_e2.T.astype(jnp.bfloat16)
    ea_bf = edge_attr.astype(jnp.bfloat16)
    edge_h_new = _edge_mlp(hsd, ea_bf, edge_h, w1s, ct, w2t,
                           b_e1.reshape(1, -1), b_e2.reshape(1, -1),
                           ln_e_g.reshape(1, -1), ln_e_b.reshape(1, -1))

    zeros_pad = jnp.zeros((N_NODES, H), jnp.float32)
    parts = _sc_scatter(edge_h_new, dst3, zeros_pad)
    p0 = parts[0]
    p1 = parts[1]

    dt = W_n1[:, :H].T.astype(jnp.bfloat16)
    et = W_n1[:, H:].T.astype(jnp.bfloat16)
    wn2t = W_n2.T.astype(jnp.bfloat16)
    node_h_new = _node_mlp(node_h, p0, p1, dt, et, wn2t,
                           b_n1.reshape(1, -1), b_n2.reshape(1, -1),
                           ln_n_g.reshape(1, -1), ln_n_b.reshape(1, -1))
    return (node_h_new, edge_h_new)
